# baseline (speedup 1.0000x reference)
import functools

import jax
import jax.numpy as jnp
from jax import lax
from jax.experimental import pallas as pl
from jax.experimental.pallas import tpu as pltpu
from jax.experimental.pallas import tpu_sc as plsc

NC = 2
NS = 16
NW = NC * NS


def _ln(x, eps=1e-5):
    m = jnp.mean(x, axis=-1, keepdims=True)
    v = jnp.mean((x - m) ** 2, axis=-1, keepdims=True)
    return (x - m) * lax.rsqrt(v + eps)


def _dotT(x, w):
    return lax.dot_general(x, w, (((1,), (1,)), ((), ())),
                           preferred_element_type=jnp.float32)


def _pack2(a, b):
    ua = lax.bitcast_convert_type(a, jnp.uint32)
    ub = lax.bitcast_convert_type(b, jnp.uint32)
    pa = (ua + jnp.uint32(0x7FFF) + ((ua >> 16) & jnp.uint32(1))) >> 16
    pb = (ub + jnp.uint32(0x7FFF) + ((ub >> 16) & jnp.uint32(1))) >> 16
    return lax.bitcast_convert_type(pa | (pb << 16), jnp.int32)


def _unpack2(w):
    u = lax.bitcast_convert_type(w, jnp.uint32)
    a = lax.bitcast_convert_type(u << 16, jnp.float32)
    b = lax.bitcast_convert_type(u & jnp.uint32(0xFFFF0000), jnp.float32)
    return a, b


def _transform_body(x_ref, w_ref, b_ref, *o_refs):
    y = _dotT(_ln(x_ref[...]), w_ref[...]) + b_ref[...]
    h = y.shape[1] // 2
    for o_ref in o_refs:
        if o_ref.dtype == jnp.int32:
            o_ref[...] = _pack2(y[:, :h], y[:, h:])
        else:
            o_ref[...] = y.astype(o_ref.dtype)


def _transform(x, w, b, blk, dtypes=(jnp.float32,), row0=0, nrows=None):
    n, d = x.shape
    nr = n if nrows is None else nrows
    ob = row0 // blk

    def owidth(dt):
        return d // 2 if dt == jnp.int32 else d

    outs = pl.pallas_call(
        _transform_body,
        grid=(nr // blk,),
        in_specs=[pl.BlockSpec((blk, d), lambda i, ob=ob: (i + ob, 0)),
                  pl.BlockSpec((d, d), lambda i: (0, 0)),
                  pl.BlockSpec((1, d), lambda i: (0, 0))],
        out_specs=[pl.BlockSpec((blk, owidth(dt)), lambda i: (i, 0))
                   for dt in dtypes],
        out_shape=[jax.ShapeDtypeStruct((nr, owidth(dt)), dt)
                   for dt in dtypes],
    )(x, w, b.reshape(1, d))
    return outs


def _joint_body(a_ref, e_ref, c_ref, w_ref, b_ref, o_ref):
    alo, ahi = _unpack2(a_ref[...])
    elo, ehi = _unpack2(e_ref[...])
    clo, chi = _unpack2(c_ref[...])
    g = jnp.concatenate([alo + elo + clo, ahi + ehi + chi], axis=1)
    g = _ln(jnp.maximum(g, 0.0))
    o_ref[...] = _ln(_dotT(g, w_ref[...]) + b_ref[...])


def _joint(ins, w, b, blk, nrows):
    hd = ins[0][0].shape[1]
    d = 2 * hd

    def spec(off):
        ob = off // blk
        return pl.BlockSpec((blk, hd), lambda i, ob=ob: (i + ob, 0))

    return pl.pallas_call(
        _joint_body,
        grid=(nrows // blk,),
        in_specs=[spec(o) for _, o in ins] +
                 [pl.BlockSpec((d, d), lambda i: (0, 0)),
                  pl.BlockSpec((1, d), lambda i: (0, 0))],
        out_specs=pl.BlockSpec((blk, d), lambda i: (i, 0)),
        out_shape=jax.ShapeDtypeStruct((nrows, d), jnp.float32),
    )(*[a for a, _ in ins], w, b.reshape(1, d))


def _merge_body(base_ref, *rest):
    aggs, (w_ref, b_ref, o_ref) = rest[:-3], rest[-3:]
    d = base_ref.shape[1]
    agg = aggs[0][...]
    for a in aggs[1:]:
        agg = agg + a[...]
    h = (_dotT(base_ref[...], w_ref[:, :d]) +
         _dotT(agg, w_ref[:, d:]) + b_ref[...])
    o_ref[...] = base_ref[...] + _ln(jnp.maximum(h, 0.0))


def _merge(base, aggs, w, b, blk):
    n, d = base.shape
    return pl.pallas_call(
        _merge_body,
        grid=(n // blk,),
        in_specs=[pl.BlockSpec((blk, d), lambda i: (i, 0))] +
                 [pl.BlockSpec((blk, d), lambda i: (i, 0)) for _ in aggs] +
                 [pl.BlockSpec((d, 2 * d), lambda i: (0, 0)),
                  pl.BlockSpec((1, d), lambda i: (0, 0))],
        out_specs=pl.BlockSpec((blk, d), lambda i: (i, 0)),
        out_shape=jax.ShapeDtypeStruct((n, d), jnp.float32),
    )(base, *aggs, w, b.reshape(1, d))


def _sc_gather(*pairs, row0=0, nrows=None):
    np_ = len(pairs)
    n, d = pairs[0][0].shape
    ne = nrows if nrows is not None else pairs[0][1].shape[0]
    dt = pairs[0][0].dtype
    chunk = 128
    nchunks = ne // chunk
    per_w = nchunks // NW
    extra = nchunks - per_w * NW
    ntask = np_ * per_w
    mesh = plsc.VectorSubcoreMesh(core_axis_name="c", subcore_axis_name="s")

    nbuf = 3

    @functools.partial(
        pl.kernel, mesh=mesh,
        out_type=[jax.ShapeDtypeStruct((ne, d), dt) for _ in pairs],
        scratch_types=[pltpu.VMEM((np_, per_w * chunk), jnp.int32),
                       pltpu.VMEM((chunk,), jnp.int32),
                       pltpu.VMEM((nbuf, chunk, d), dt)]
                      + [pltpu.SemaphoreType.DMA] * (2 * nbuf + np_),
    )
    def k(*args):
        tabs = args[:2 * np_:2]
        idxs = args[1:2 * np_:2]
        outs = args[2 * np_:3 * np_]
        idx_all, idx_x, buf_v = args[3 * np_:3 * np_ + 3]
        sems = args[3 * np_ + 3:]
        gsem, wsem, isem = sems[:nbuf], sems[nbuf:2 * nbuf], sems[2 * nbuf:]
        wid = lax.axis_index("s") * NC + lax.axis_index("c")
        ih = [pltpu.async_copy(
                  idxs[p].at[pl.ds(row0 + wid * per_w * chunk,
                                   per_w * chunk)],
                  idx_all.at[p], isem[p]) for p in range(np_)]

        def start_gather(t, sl):
            p, i = t % np_, t // np_
            return pltpu.async_copy(
                tabs[p].at[idx_all.at[p].at[pl.ds(i * chunk, chunk)]],
                buf_v.at[sl], gsem[sl])

        def start_write(t, sl):
            p, i = t % np_, t // np_
            base = (wid * per_w + i) * chunk
            return pltpu.async_copy(buf_v.at[sl],
                                    outs[p].at[pl.ds(base, chunk), :],
                                    wsem[sl])

        for h in ih:
            h.wait()
        gh, wh = {}, {}
        for j in range(min(nbuf - 1, ntask)):
            gh[j % nbuf] = start_gather(j, j % nbuf)
        for t in range(ntask):
            sl = t % nbuf
            nx = t + nbuf - 1
            if nx < ntask:
                tt = nx % nbuf
                if tt in wh:
                    wh[tt].wait()
                gh[tt] = start_gather(nx, tt)
            gh[sl].wait()
            wh[sl] = start_write(t, sl)
        for sl in wh:
            wh[sl].wait()

        if extra:
            @pl.when(wid < extra)
            def _():
                base = (NW * per_w + wid) * chunk
                for p in range(np_):
                    pltpu.sync_copy(idxs[p].at[pl.ds(row0 + base, chunk)],
                                    idx_x)
                    pltpu.async_copy(tabs[p].at[idx_x], buf_v.at[0],
                                     gsem[0]).wait()
                    pltpu.sync_copy(buf_v.at[0],
                                    outs[p].at[pl.ds(base, chunk), :])

    flat = []
    for t, i in pairs:
        flat += [t, i]
    res = k(*flat)
    return tuple(res) if isinstance(res, (list, tuple)) else (res,)


def _sc_segsum(joint, idx, nseg, row0=0):
    ne, d = joint.shape
    dh = d // NC
    chunk = 128
    nchunks = ne // chunk
    per_s = nchunks // NS
    extra = nchunks - per_s * NS
    rows_per_s = (nseg // NS) // 8 * 8
    tail = nseg - rows_per_s * NS
    zr = 16
    mesh = plsc.VectorSubcoreMesh(core_axis_name="c", subcore_axis_name="s")

    @functools.partial(
        pl.kernel, mesh=mesh,
        out_type=jax.ShapeDtypeStruct((nseg, d), jnp.float32),
        scratch_types=[pltpu.VMEM((chunk,), jnp.int32),
                       pltpu.VMEM((chunk,), jnp.int32),
                       pltpu.VMEM((2, chunk, dh), jnp.float32),
                       pltpu.VMEM((zr, dh), jnp.float32),
                       pltpu.VMEM_SHARED((nseg, dh), jnp.float32)]
                      + [pltpu.SemaphoreType.DMA] * 6,
    )
    def k(joint_hbm, idx_hbm, out_hbm, idx_a, idx_b, buf_v, zbuf,
          acc_sh, *sems):
        nbuf = 2
        c = lax.axis_index("c")
        s = lax.axis_index("s")
        idxr = (idx_a, idx_b)
        isem, rsem, ssem = sems[:2], sems[2:4], sems[4:]
        row_base = s * rows_per_s
        nz = (rows_per_s + jnp.where(s == NS - 1, tail, 0)) // zr

        def zrow(r, carry):
            def zcol(q, carry2):
                zbuf[r, pl.ds(q * 16, 16)] = jnp.zeros((16,), jnp.float32)
                return carry2
            return lax.fori_loop(0, dh // 16, zcol, carry)
        lax.fori_loop(0, zr, zrow, 0)

        def zdma(t, carry):
            pltpu.sync_copy(zbuf, acc_sh.at[pl.ds(row_base + t * zr, zr)])
            return carry
        lax.fori_loop(0, nz, zdma, 0)
        plsc.subcore_barrier()

        def start_load(i, sl):
            base = (s * per_s + i) * chunk
            ih = pltpu.async_copy(idx_hbm.at[pl.ds(row0 + base, chunk)],
                                  idxr[sl], isem[sl])
            rh = pltpu.async_copy(
                joint_hbm.at[pl.ds(base, chunk), pl.ds(c * dh, dh)],
                buf_v.at[sl], rsem[sl])
            return ih, rh

        def start_scatter(sl):
            return pltpu.async_copy(buf_v.at[sl], acc_sh.at[idxr[sl]],
                                    ssem[sl], add=True)

        lh, sh = {}, {}
        for j in range(min(nbuf - 1, per_s)):
            lh[j % nbuf] = start_load(j, j % nbuf)
        for i in range(per_s):
            sl = i % nbuf
            nx = i + nbuf - 1
            if nx < per_s:
                t = nx % nbuf
                if t in sh:
                    sh[t].wait()
                lh[t] = start_load(nx, t)
            lh[sl][0].wait()
            lh[sl][1].wait()
            sh[sl] = start_scatter(sl)
        for sl in sh:
            sh[sl].wait()

        if extra:
            @pl.when(s < extra)
            def _():
                base = (NS * per_s + s) * chunk
                pltpu.sync_copy(idx_hbm.at[pl.ds(row0 + base, chunk)], idx_a)
                pltpu.sync_copy(
                    joint_hbm.at[pl.ds(base, chunk), pl.ds(c * dh, dh)],
                    buf_v.at[0])
                pltpu.sync_copy(buf_v.at[0], acc_sh.at[idx_a], add=True)
        plsc.subcore_barrier()

        pltpu.sync_copy(
            acc_sh.at[pl.ds(row_base, rows_per_s)],
            out_hbm.at[pl.ds(row_base, rows_per_s), pl.ds(c * dh, dh)])

        if tail:
            @pl.when(s == NS - 1)
            def _():
                tb = NS * rows_per_s
                pltpu.sync_copy(
                    acc_sh.at[pl.ds(tb, tail)],
                    out_hbm.at[pl.ds(tb, tail), pl.ds(c * dh, dh)])

    return k(joint, idx)


def kernel(variable_emb, edge_emb, constraint_emb, W_left, b_left, W_edge,
           W_right, W_join, b_join, W_merge, b_merge, e_u, e_v):
    nu, d = variable_emb.shape
    nv = constraint_emb.shape[0]
    zb = jnp.zeros((d,), jnp.float32)

    pk = jnp.int32

    var_t, var_tb = _transform(variable_emb, W_left, b_left, 2000,
                               (jnp.float32, pk))
    (edge_tb,) = _transform(edge_emb, W_edge, zb, 1000, (pk,))
    con_t, con_tb = _transform(constraint_emb, W_right, zb, 2000,
                               (jnp.float32, pk))

    ne = e_u.shape[0]
    u = ne // 10
    sizes = [u, 4 * u, 4 * u, u]
    offs = [0, u, 5 * u, 9 * u]
    pieces = list(zip(offs, sizes))

    et, va, ca1, agg1 = {}, {}, {}, []
    for p, (off, ln) in enumerate(pieces):
        (et[p],) = _transform(edge_emb, W_edge, zb, 1000, (pk,),
                              row0=off, nrows=ln)
        va[p], ca1[p] = _sc_gather((var_tb, e_u), (con_tb, e_v),
                                   row0=off, nrows=ln)
    for p, (off, ln) in enumerate(pieces):
        j = _joint([(va[p], 0), (et[p], 0), (ca1[p], 0)],
                   W_join, b_join, 1000, ln)
        agg1.append(_sc_segsum(j, e_v, nv, off))
    con2 = _merge(con_t, tuple(agg1), W_merge, b_merge, 2000)

    (con_t2b,) = _transform(con2, W_right, zb, 2000, (pk,))
    ca2 = {}
    for p, (off, ln) in enumerate(pieces):
        (ca2[p],) = _sc_gather((con_t2b, e_v), row0=off, nrows=ln)
    agg2 = []
    for p, (off, ln) in enumerate(pieces):
        j = _joint([(va[p], 0), (et[p], 0), (ca2[p], 0)],
                   W_join, b_join, 1000, ln)
        agg2.append(_sc_segsum(j, e_u, nu, off))
    var2 = _merge(var_t, tuple(agg2), W_merge, b_merge, 2000)

    return (var2, con2)


# speedup vs baseline: 1.0015x; 1.0015x over previous
import functools

import jax
import jax.numpy as jnp
from jax import lax
from jax.experimental import pallas as pl
from jax.experimental.pallas import tpu as pltpu
from jax.experimental.pallas import tpu_sc as plsc

NC = 2
NS = 16
NW = NC * NS


def _ln(x, eps=1e-5):
    m = jnp.mean(x, axis=-1, keepdims=True)
    v = jnp.mean((x - m) ** 2, axis=-1, keepdims=True)
    return (x - m) * lax.rsqrt(v + eps)


def _dotT(x, w):
    return lax.dot_general(x, w, (((1,), (1,)), ((), ())),
                           preferred_element_type=jnp.float32)


def _pack2(a, b):
    ua = lax.bitcast_convert_type(a, jnp.uint32)
    ub = lax.bitcast_convert_type(b, jnp.uint32)
    pa = (ua + jnp.uint32(0x7FFF) + ((ua >> 16) & jnp.uint32(1))) >> 16
    pb = (ub + jnp.uint32(0x7FFF) + ((ub >> 16) & jnp.uint32(1))) >> 16
    return lax.bitcast_convert_type(pa | (pb << 16), jnp.int32)


def _unpack2(w):
    u = lax.bitcast_convert_type(w, jnp.uint32)
    a = lax.bitcast_convert_type(u << 16, jnp.float32)
    b = lax.bitcast_convert_type(u & jnp.uint32(0xFFFF0000), jnp.float32)
    return a, b


def _transform_body(x_ref, w_ref, b_ref, *o_refs):
    y = _dotT(_ln(x_ref[...]), w_ref[...]) + b_ref[...]
    h = y.shape[1] // 2
    for o_ref in o_refs:
        if o_ref.dtype == jnp.int32:
            o_ref[...] = _pack2(y[:, :h], y[:, h:])
        else:
            o_ref[...] = y.astype(o_ref.dtype)


def _transform(x, w, b, blk, dtypes=(jnp.float32,), row0=0, nrows=None):
    n, d = x.shape
    nr = n if nrows is None else nrows
    ob = row0 // blk

    def owidth(dt):
        return d // 2 if dt == jnp.int32 else d

    outs = pl.pallas_call(
        _transform_body,
        grid=(nr // blk,),
        in_specs=[pl.BlockSpec((blk, d), lambda i, ob=ob: (i + ob, 0)),
                  pl.BlockSpec((d, d), lambda i: (0, 0)),
                  pl.BlockSpec((1, d), lambda i: (0, 0))],
        out_specs=[pl.BlockSpec((blk, owidth(dt)), lambda i: (i, 0))
                   for dt in dtypes],
        out_shape=[jax.ShapeDtypeStruct((nr, owidth(dt)), dt)
                   for dt in dtypes],
    )(x, w, b.reshape(1, d))
    return outs


def _joint_body(a_ref, e_ref, c_ref, w_ref, b_ref, o_ref):
    alo, ahi = _unpack2(a_ref[...])
    elo, ehi = _unpack2(e_ref[...])
    clo, chi = _unpack2(c_ref[...])
    g = jnp.concatenate([alo + elo + clo, ahi + ehi + chi], axis=1)
    g = _ln(jnp.maximum(g, 0.0))
    o_ref[...] = _ln(_dotT(g, w_ref[...]) + b_ref[...])


def _joint(ins, w, b, blk, nrows):
    hd = ins[0][0].shape[1]
    d = 2 * hd

    def spec(off):
        ob = off // blk
        return pl.BlockSpec((blk, hd), lambda i, ob=ob: (i + ob, 0))

    return pl.pallas_call(
        _joint_body,
        grid=(nrows // blk,),
        in_specs=[spec(o) for _, o in ins] +
                 [pl.BlockSpec((d, d), lambda i: (0, 0)),
                  pl.BlockSpec((1, d), lambda i: (0, 0))],
        out_specs=pl.BlockSpec((blk, d), lambda i: (i, 0)),
        out_shape=jax.ShapeDtypeStruct((nrows, d), jnp.float32),
    )(*[a for a, _ in ins], w, b.reshape(1, d))


def _merge_body(base_ref, *rest):
    aggs, (w_ref, b_ref, o_ref) = rest[:-3], rest[-3:]
    d = base_ref.shape[1]
    agg = aggs[0][...]
    for a in aggs[1:]:
        agg = agg + a[...]
    h = (_dotT(base_ref[...], w_ref[:, :d]) +
         _dotT(agg, w_ref[:, d:]) + b_ref[...])
    o_ref[...] = base_ref[...] + _ln(jnp.maximum(h, 0.0))


def _merge(base, aggs, w, b, blk):
    n, d = base.shape
    return pl.pallas_call(
        _merge_body,
        grid=(n // blk,),
        in_specs=[pl.BlockSpec((blk, d), lambda i: (i, 0))] +
                 [pl.BlockSpec((blk, d), lambda i: (i, 0)) for _ in aggs] +
                 [pl.BlockSpec((d, 2 * d), lambda i: (0, 0)),
                  pl.BlockSpec((1, d), lambda i: (0, 0))],
        out_specs=pl.BlockSpec((blk, d), lambda i: (i, 0)),
        out_shape=jax.ShapeDtypeStruct((n, d), jnp.float32),
    )(base, *aggs, w, b.reshape(1, d))


def _sc_gather(*pairs, row0=0, nrows=None):
    np_ = len(pairs)
    n, d = pairs[0][0].shape
    ne = nrows if nrows is not None else pairs[0][1].shape[0]
    dt = pairs[0][0].dtype
    chunk = 128
    nchunks = ne // chunk
    per_w = nchunks // NW
    extra = nchunks - per_w * NW
    ntask = np_ * per_w
    mesh = plsc.VectorSubcoreMesh(core_axis_name="c", subcore_axis_name="s")

    nbuf = 3

    @functools.partial(
        pl.kernel, mesh=mesh,
        out_type=[jax.ShapeDtypeStruct((ne, d), dt) for _ in pairs],
        scratch_types=[pltpu.VMEM((np_, per_w * chunk), jnp.int32),
                       pltpu.VMEM((chunk,), jnp.int32),
                       pltpu.VMEM((nbuf, chunk, d), dt)]
                      + [pltpu.SemaphoreType.DMA] * (2 * nbuf + np_),
    )
    def k(*args):
        tabs = args[:2 * np_:2]
        idxs = args[1:2 * np_:2]
        outs = args[2 * np_:3 * np_]
        idx_all, idx_x, buf_v = args[3 * np_:3 * np_ + 3]
        sems = args[3 * np_ + 3:]
        gsem, wsem, isem = sems[:nbuf], sems[nbuf:2 * nbuf], sems[2 * nbuf:]
        wid = lax.axis_index("s") * NC + lax.axis_index("c")
        ih = [pltpu.async_copy(
                  idxs[p].at[pl.ds(row0 + wid * per_w * chunk,
                                   per_w * chunk)],
                  idx_all.at[p], isem[p]) for p in range(np_)]

        def start_gather(t, sl):
            p, i = t % np_, t // np_
            return pltpu.async_copy(
                tabs[p].at[idx_all.at[p].at[pl.ds(i * chunk, chunk)]],
                buf_v.at[sl], gsem[sl])

        def start_write(t, sl):
            p, i = t % np_, t // np_
            base = (wid * per_w + i) * chunk
            return pltpu.async_copy(buf_v.at[sl],
                                    outs[p].at[pl.ds(base, chunk), :],
                                    wsem[sl])

        for h in ih:
            h.wait()
        gh, wh = {}, {}
        for j in range(min(nbuf - 1, ntask)):
            gh[j % nbuf] = start_gather(j, j % nbuf)
        for t in range(ntask):
            sl = t % nbuf
            nx = t + nbuf - 1
            if nx < ntask:
                tt = nx % nbuf
                if tt in wh:
                    wh[tt].wait()
                gh[tt] = start_gather(nx, tt)
            gh[sl].wait()
            wh[sl] = start_write(t, sl)
        for sl in wh:
            wh[sl].wait()

        if extra:
            @pl.when(wid < extra)
            def _():
                base = (NW * per_w + wid) * chunk
                for p in range(np_):
                    pltpu.sync_copy(idxs[p].at[pl.ds(row0 + base, chunk)],
                                    idx_x)
                    pltpu.async_copy(tabs[p].at[idx_x], buf_v.at[0],
                                     gsem[0]).wait()
                    pltpu.sync_copy(buf_v.at[0],
                                    outs[p].at[pl.ds(base, chunk), :])

    flat = []
    for t, i in pairs:
        flat += [t, i]
    res = k(*flat)
    return tuple(res) if isinstance(res, (list, tuple)) else (res,)


def _sc_segsum(joint, idx, nseg, row0=0):
    ne, d = joint.shape
    dh = d // NC
    chunk = 128
    nchunks = ne // chunk
    per_s = nchunks // NS
    extra = nchunks - per_s * NS
    rows_per_s = (nseg // NS) // 8 * 8
    tail = nseg - rows_per_s * NS
    zr = 16
    mesh = plsc.VectorSubcoreMesh(core_axis_name="c", subcore_axis_name="s")

    @functools.partial(
        pl.kernel, mesh=mesh,
        out_type=jax.ShapeDtypeStruct((nseg, d), jnp.float32),
        scratch_types=[pltpu.VMEM((chunk,), jnp.int32),
                       pltpu.VMEM((chunk,), jnp.int32),
                       pltpu.VMEM((2, chunk, dh), jnp.float32),
                       pltpu.VMEM((zr, dh), jnp.float32),
                       pltpu.VMEM_SHARED((nseg, dh), jnp.float32)]
                      + [pltpu.SemaphoreType.DMA] * 6,
    )
    def k(joint_hbm, idx_hbm, out_hbm, idx_a, idx_b, buf_v, zbuf,
          acc_sh, *sems):
        nbuf = 2
        c = lax.axis_index("c")
        s = lax.axis_index("s")
        idxr = (idx_a, idx_b)
        isem, rsem, ssem = sems[:2], sems[2:4], sems[4:]
        row_base = s * rows_per_s
        nz = (rows_per_s + jnp.where(s == NS - 1, tail, 0)) // zr

        def zrow(r, carry):
            def zcol(q, carry2):
                zbuf[r, pl.ds(q * 16, 16)] = jnp.zeros((16,), jnp.float32)
                return carry2
            return lax.fori_loop(0, dh // 16, zcol, carry)
        lax.fori_loop(0, zr, zrow, 0)

        def zdma(t, carry):
            pltpu.sync_copy(zbuf, acc_sh.at[pl.ds(row_base + t * zr, zr)])
            return carry
        lax.fori_loop(0, nz, zdma, 0)
        plsc.subcore_barrier()

        def start_load(i, sl):
            base = (s * per_s + i) * chunk
            ih = pltpu.async_copy(idx_hbm.at[pl.ds(row0 + base, chunk)],
                                  idxr[sl], isem[sl])
            rh = pltpu.async_copy(
                joint_hbm.at[pl.ds(base, chunk), pl.ds(c * dh, dh)],
                buf_v.at[sl], rsem[sl])
            return ih, rh

        def start_scatter(sl):
            return pltpu.async_copy(buf_v.at[sl], acc_sh.at[idxr[sl]],
                                    ssem[sl], add=True)

        lh, sh = {}, {}
        for j in range(min(nbuf - 1, per_s)):
            lh[j % nbuf] = start_load(j, j % nbuf)
        for i in range(per_s):
            sl = i % nbuf
            nx = i + nbuf - 1
            if nx < per_s:
                t = nx % nbuf
                if t in sh:
                    sh[t].wait()
                lh[t] = start_load(nx, t)
            lh[sl][0].wait()
            lh[sl][1].wait()
            sh[sl] = start_scatter(sl)
        for sl in sh:
            sh[sl].wait()

        if extra:
            @pl.when(s < extra)
            def _():
                base = (NS * per_s + s) * chunk
                pltpu.sync_copy(idx_hbm.at[pl.ds(row0 + base, chunk)], idx_a)
                pltpu.sync_copy(
                    joint_hbm.at[pl.ds(base, chunk), pl.ds(c * dh, dh)],
                    buf_v.at[0])
                pltpu.sync_copy(buf_v.at[0], acc_sh.at[idx_a], add=True)
        plsc.subcore_barrier()

        pltpu.sync_copy(
            acc_sh.at[pl.ds(row_base, rows_per_s)],
            out_hbm.at[pl.ds(row_base, rows_per_s), pl.ds(c * dh, dh)])

        if tail:
            @pl.when(s == NS - 1)
            def _():
                tb = NS * rows_per_s
                pltpu.sync_copy(
                    acc_sh.at[pl.ds(tb, tail)],
                    out_hbm.at[pl.ds(tb, tail), pl.ds(c * dh, dh)])

    return k(joint, idx)


def kernel(variable_emb, edge_emb, constraint_emb, W_left, b_left, W_edge,
           W_right, W_join, b_join, W_merge, b_merge, e_u, e_v):
    nu, d = variable_emb.shape
    nv = constraint_emb.shape[0]
    zb = jnp.zeros((d,), jnp.float32)

    pk = jnp.int32

    var_t, var_tb = _transform(variable_emb, W_left, b_left, 2000,
                               (jnp.float32, pk))
    (edge_tb,) = _transform(edge_emb, W_edge, zb, 1000, (pk,))
    con_t, con_tb = _transform(constraint_emb, W_right, zb, 2000,
                               (jnp.float32, pk))

    ne = e_u.shape[0]
    u = ne // 10
    sizes = [4 * u, 5 * u, u]
    offs = [0, 4 * u, 9 * u]
    pieces = list(zip(offs, sizes))

    et, va, ca1, agg1 = {}, {}, {}, []
    for p, (off, ln) in enumerate(pieces):
        (et[p],) = _transform(edge_emb, W_edge, zb, 1000, (pk,),
                              row0=off, nrows=ln)
        va[p], ca1[p] = _sc_gather((var_tb, e_u), (con_tb, e_v),
                                   row0=off, nrows=ln)
    for p, (off, ln) in enumerate(pieces):
        j = _joint([(va[p], 0), (et[p], 0), (ca1[p], 0)],
                   W_join, b_join, 1000, ln)
        agg1.append(_sc_segsum(j, e_v, nv, off))
    con2 = _merge(con_t, tuple(agg1), W_merge, b_merge, 2000)

    (con_t2b,) = _transform(con2, W_right, zb, 2000, (pk,))
    order = [2, 1, 0]
    ca2, agg2 = {}, {}
    for p in order:
        off, ln = pieces[p]
        (ca2[p],) = _sc_gather((con_t2b, e_v), row0=off, nrows=ln)
    for p in order:
        off, ln = pieces[p]
        j = _joint([(va[p], 0), (et[p], 0), (ca2[p], 0)],
                   W_join, b_join, 1000, ln)
        agg2[p] = _sc_segsum(j, e_u, nu, off)
    var2 = _merge(var_t, tuple(agg2[p] for p in order), W_merge, b_merge,
                  2000)

    return (var2, con2)


# speedup vs baseline: 1.0212x; 1.0197x over previous
import functools

import jax
import jax.numpy as jnp
from jax import lax
from jax.experimental import pallas as pl
from jax.experimental.pallas import tpu as pltpu
from jax.experimental.pallas import tpu_sc as plsc

NC = 2
NS = 16
NW = NC * NS


def _ln(x, eps=1e-5):
    m = jnp.mean(x, axis=-1, keepdims=True)
    v = jnp.mean((x - m) ** 2, axis=-1, keepdims=True)
    return (x - m) * lax.rsqrt(v + eps)


def _dotT(x, w):
    return lax.dot_general(x, w, (((1,), (1,)), ((), ())),
                           preferred_element_type=jnp.float32)


def _pack2(a, b):
    ua = lax.bitcast_convert_type(a, jnp.uint32)
    ub = lax.bitcast_convert_type(b, jnp.uint32)
    pa = (ua + jnp.uint32(0x7FFF) + ((ua >> 16) & jnp.uint32(1))) >> 16
    pb = (ub + jnp.uint32(0x7FFF) + ((ub >> 16) & jnp.uint32(1))) >> 16
    return lax.bitcast_convert_type(pa | (pb << 16), jnp.int32)


def _unpack2(w):
    u = lax.bitcast_convert_type(w, jnp.uint32)
    a = lax.bitcast_convert_type(u << 16, jnp.float32)
    b = lax.bitcast_convert_type(u & jnp.uint32(0xFFFF0000), jnp.float32)
    return a, b


def _transform_body(x_ref, w_ref, b_ref, *o_refs):
    y = _dotT(_ln(x_ref[...]), w_ref[...]) + b_ref[...]
    h = y.shape[1] // 2
    for o_ref in o_refs:
        if o_ref.dtype == jnp.int32:
            o_ref[...] = _pack2(y[:, :h], y[:, h:])
        else:
            o_ref[...] = y.astype(o_ref.dtype)


def _transform(x, w, b, blk, dtypes=(jnp.float32,), row0=0, nrows=None):
    n, d = x.shape
    nr = n if nrows is None else nrows
    ob = row0 // blk

    def owidth(dt):
        return d // 2 if dt == jnp.int32 else d

    outs = pl.pallas_call(
        _transform_body,
        grid=(nr // blk,),
        in_specs=[pl.BlockSpec((blk, d), lambda i, ob=ob: (i + ob, 0)),
                  pl.BlockSpec((d, d), lambda i: (0, 0)),
                  pl.BlockSpec((1, d), lambda i: (0, 0))],
        out_specs=[pl.BlockSpec((blk, owidth(dt)), lambda i: (i, 0))
                   for dt in dtypes],
        out_shape=[jax.ShapeDtypeStruct((nr, owidth(dt)), dt)
                   for dt in dtypes],
    )(x, w, b.reshape(1, d))
    return outs


def _joint_body(a_ref, e_ref, c_ref, w_ref, b_ref, o_ref):
    alo, ahi = _unpack2(a_ref[...])
    elo, ehi = _unpack2(e_ref[...])
    clo, chi = _unpack2(c_ref[...])
    g = jnp.concatenate([alo + elo + clo, ahi + ehi + chi], axis=1)
    g = _ln(jnp.maximum(g, 0.0))
    o_ref[...] = _ln(_dotT(g, w_ref[...]) + b_ref[...])


def _joint(ins, w, b, blk, nrows):
    hd = ins[0][0].shape[1]
    d = 2 * hd

    def spec(off):
        ob = off // blk
        return pl.BlockSpec((blk, hd), lambda i, ob=ob: (i + ob, 0))

    return pl.pallas_call(
        _joint_body,
        grid=(nrows // blk,),
        in_specs=[spec(o) for _, o in ins] +
                 [pl.BlockSpec((d, d), lambda i: (0, 0)),
                  pl.BlockSpec((1, d), lambda i: (0, 0))],
        out_specs=pl.BlockSpec((blk, d), lambda i: (i, 0)),
        out_shape=jax.ShapeDtypeStruct((nrows, d), jnp.float32),
    )(*[a for a, _ in ins], w, b.reshape(1, d))


def _merge_body(base_ref, *rest):
    aggs, (w_ref, b_ref, o_ref) = rest[:-3], rest[-3:]
    d = base_ref.shape[1]
    agg = aggs[0][...]
    for a in aggs[1:]:
        agg = agg + a[...]
    h = (_dotT(base_ref[...], w_ref[:, :d]) +
         _dotT(agg, w_ref[:, d:]) + b_ref[...])
    o_ref[...] = base_ref[...] + _ln(jnp.maximum(h, 0.0))


def _merge(base, aggs, w, b, blk):
    n, d = base.shape
    return pl.pallas_call(
        _merge_body,
        grid=(n // blk,),
        in_specs=[pl.BlockSpec((blk, d), lambda i: (i, 0))] +
                 [pl.BlockSpec((blk, d), lambda i: (i, 0)) for _ in aggs] +
                 [pl.BlockSpec((d, 2 * d), lambda i: (0, 0)),
                  pl.BlockSpec((1, d), lambda i: (0, 0))],
        out_specs=pl.BlockSpec((blk, d), lambda i: (i, 0)),
        out_shape=jax.ShapeDtypeStruct((n, d), jnp.float32),
    )(base, *aggs, w, b.reshape(1, d))


def _sc_gather(*pairs, row0=0, nrows=None):
    np_ = len(pairs)
    n, d = pairs[0][0].shape
    ne = nrows if nrows is not None else pairs[0][1].shape[0]
    dt = pairs[0][0].dtype
    chunk = 128
    nchunks = ne // chunk
    per_w = nchunks // NW
    extra = nchunks - per_w * NW
    ntask = np_ * per_w
    mesh = plsc.VectorSubcoreMesh(core_axis_name="c", subcore_axis_name="s")

    nbuf = 3

    @functools.partial(
        pl.kernel, mesh=mesh,
        out_type=[jax.ShapeDtypeStruct((ne, d), dt) for _ in pairs],
        scratch_types=[pltpu.VMEM((np_, per_w * chunk), jnp.int32),
                       pltpu.VMEM((chunk,), jnp.int32),
                       pltpu.VMEM((nbuf, chunk, d), dt)]
                      + [pltpu.SemaphoreType.DMA] * (2 * nbuf + np_),
    )
    def k(*args):
        tabs = args[:2 * np_:2]
        idxs = args[1:2 * np_:2]
        outs = args[2 * np_:3 * np_]
        idx_all, idx_x, buf_v = args[3 * np_:3 * np_ + 3]
        sems = args[3 * np_ + 3:]
        gsem, wsem, isem = sems[:nbuf], sems[nbuf:2 * nbuf], sems[2 * nbuf:]
        wid = lax.axis_index("s") * NC + lax.axis_index("c")
        ih = [pltpu.async_copy(
                  idxs[p].at[pl.ds(row0 + wid * per_w * chunk,
                                   per_w * chunk)],
                  idx_all.at[p], isem[p]) for p in range(np_)]

        def start_gather(t, sl):
            p, i = t % np_, t // np_
            return pltpu.async_copy(
                tabs[p].at[idx_all.at[p].at[pl.ds(i * chunk, chunk)]],
                buf_v.at[sl], gsem[sl])

        def start_write(t, sl):
            p, i = t % np_, t // np_
            base = (wid * per_w + i) * chunk
            return pltpu.async_copy(buf_v.at[sl],
                                    outs[p].at[pl.ds(base, chunk), :],
                                    wsem[sl])

        for h in ih:
            h.wait()
        gh, wh = {}, {}
        for j in range(min(nbuf - 1, ntask)):
            gh[j % nbuf] = start_gather(j, j % nbuf)
        for t in range(ntask):
            sl = t % nbuf
            nx = t + nbuf - 1
            if nx < ntask:
                tt = nx % nbuf
                if tt in wh:
                    wh[tt].wait()
                gh[tt] = start_gather(nx, tt)
            gh[sl].wait()
            wh[sl] = start_write(t, sl)
        for sl in wh:
            wh[sl].wait()

        if extra:
            @pl.when(wid < extra)
            def _():
                base = (NW * per_w + wid) * chunk
                for p in range(np_):
                    pltpu.sync_copy(idxs[p].at[pl.ds(row0 + base, chunk)],
                                    idx_x)
                    pltpu.async_copy(tabs[p].at[idx_x], buf_v.at[0],
                                     gsem[0]).wait()
                    pltpu.sync_copy(buf_v.at[0],
                                    outs[p].at[pl.ds(base, chunk), :])

    flat = []
    for t, i in pairs:
        flat += [t, i]
    res = k(*flat)
    return tuple(res) if isinstance(res, (list, tuple)) else (res,)


def _sc_segsum(joint, idx, nseg, row0=0):
    ne, d = joint.shape
    dh = d // NC
    chunk = 128
    nchunks = ne // chunk
    per_s = nchunks // NS
    extra = nchunks - per_s * NS
    rows_per_s = (nseg // NS) // 8 * 8
    tail = nseg - rows_per_s * NS
    zr = 16
    mesh = plsc.VectorSubcoreMesh(core_axis_name="c", subcore_axis_name="s")

    @functools.partial(
        pl.kernel, mesh=mesh,
        out_type=jax.ShapeDtypeStruct((nseg, d), jnp.float32),
        scratch_types=[pltpu.VMEM((chunk,), jnp.int32),
                       pltpu.VMEM((chunk,), jnp.int32),
                       pltpu.VMEM((2, chunk, dh), jnp.float32),
                       pltpu.VMEM((zr, dh), jnp.float32),
                       pltpu.VMEM_SHARED((nseg, dh), jnp.float32)]
                      + [pltpu.SemaphoreType.DMA] * 6,
    )
    def k(joint_hbm, idx_hbm, out_hbm, idx_a, idx_b, buf_v, zbuf,
          acc_sh, *sems):
        nbuf = 2
        c = lax.axis_index("c")
        s = lax.axis_index("s")
        idxr = (idx_a, idx_b)
        isem, rsem, ssem = sems[:2], sems[2:4], sems[4:]
        row_base = s * rows_per_s
        nz = (rows_per_s + jnp.where(s == NS - 1, tail, 0)) // zr

        def zrow(r, carry):
            def zcol(q, carry2):
                zbuf[r, pl.ds(q * 16, 16)] = jnp.zeros((16,), jnp.float32)
                return carry2
            return lax.fori_loop(0, dh // 16, zcol, carry)
        lax.fori_loop(0, zr, zrow, 0)

        def zdma(t, carry):
            pltpu.sync_copy(zbuf, acc_sh.at[pl.ds(row_base + t * zr, zr)])
            return carry
        lax.fori_loop(0, nz, zdma, 0)
        plsc.subcore_barrier()

        def start_load(i, sl):
            base = (s * per_s + i) * chunk
            ih = pltpu.async_copy(idx_hbm.at[pl.ds(row0 + base, chunk)],
                                  idxr[sl], isem[sl])
            rh = pltpu.async_copy(
                joint_hbm.at[pl.ds(base, chunk), pl.ds(c * dh, dh)],
                buf_v.at[sl], rsem[sl])
            return ih, rh

        def start_scatter(sl):
            return pltpu.async_copy(buf_v.at[sl], acc_sh.at[idxr[sl]],
                                    ssem[sl], add=True)

        lh, sh = {}, {}
        for j in range(min(nbuf - 1, per_s)):
            lh[j % nbuf] = start_load(j, j % nbuf)
        for i in range(per_s):
            sl = i % nbuf
            nx = i + nbuf - 1
            if nx < per_s:
                t = nx % nbuf
                if t in sh:
                    sh[t].wait()
                lh[t] = start_load(nx, t)
            lh[sl][0].wait()
            lh[sl][1].wait()
            sh[sl] = start_scatter(sl)
        for sl in sh:
            sh[sl].wait()

        if extra:
            @pl.when(s < extra)
            def _():
                base = (NS * per_s + s) * chunk
                pltpu.sync_copy(idx_hbm.at[pl.ds(row0 + base, chunk)], idx_a)
                pltpu.sync_copy(
                    joint_hbm.at[pl.ds(base, chunk), pl.ds(c * dh, dh)],
                    buf_v.at[0])
                pltpu.sync_copy(buf_v.at[0], acc_sh.at[idx_a], add=True)
        plsc.subcore_barrier()

        pltpu.sync_copy(
            acc_sh.at[pl.ds(row_base, rows_per_s)],
            out_hbm.at[pl.ds(row_base, rows_per_s), pl.ds(c * dh, dh)])

        if tail:
            @pl.when(s == NS - 1)
            def _():
                tb = NS * rows_per_s
                pltpu.sync_copy(
                    acc_sh.at[pl.ds(tb, tail)],
                    out_hbm.at[pl.ds(tb, tail), pl.ds(c * dh, dh)])

    return k(joint, idx)


def kernel(variable_emb, edge_emb, constraint_emb, W_left, b_left, W_edge,
           W_right, W_join, b_join, W_merge, b_merge, e_u, e_v):
    nu, d = variable_emb.shape
    nv = constraint_emb.shape[0]
    zb = jnp.zeros((d,), jnp.float32)

    pk = jnp.int32

    var_t, var_tb = _transform(variable_emb, W_left, b_left, 2000,
                               (jnp.float32, pk))
    (edge_tb,) = _transform(edge_emb, W_edge, zb, 1000, (pk,))
    con_t, con_tb = _transform(constraint_emb, W_right, zb, 2000,
                               (jnp.float32, pk))

    ne = e_u.shape[0]
    u = ne // 10
    sizes = [4 * u, 4 * u, 2 * u]
    offs = [0, 4 * u, 8 * u]
    pieces = list(zip(offs, sizes))

    et, va, ca1, agg1 = {}, {}, {}, []
    for p, (off, ln) in enumerate(pieces):
        (et[p],) = _transform(edge_emb, W_edge, zb, 1000, (pk,),
                              row0=off, nrows=ln)
        va[p], ca1[p] = _sc_gather((var_tb, e_u), (con_tb, e_v),
                                   row0=off, nrows=ln)
    for p, (off, ln) in enumerate(pieces):
        j = _joint([(va[p], 0), (et[p], 0), (ca1[p], 0)],
                   W_join, b_join, 1000, ln)
        agg1.append(_sc_segsum(j, e_v, nv, off))
    con2 = _merge(con_t, tuple(agg1), W_merge, b_merge, 2000)

    (con_t2b,) = _transform(con2, W_right, zb, 2000, (pk,))
    order = [2, 0, 1]
    ca2, agg2 = {}, {}
    for p in order:
        off, ln = pieces[p]
        (ca2[p],) = _sc_gather((con_t2b, e_v), row0=off, nrows=ln)
    for p in order:
        off, ln = pieces[p]
        j = _joint([(va[p], 0), (et[p], 0), (ca2[p], 0)],
                   W_join, b_join, 1000, ln)
        agg2[p] = _sc_segsum(j, e_u, nu, off)
    var2 = _merge(var_t, tuple(agg2[p] for p in order), W_merge, b_merge,
                  2000)

    return (var2, con2)


# speedup vs baseline: 1.0273x; 1.0060x over previous
import functools

import jax
import jax.numpy as jnp
from jax import lax
from jax.experimental import pallas as pl
from jax.experimental.pallas import tpu as pltpu
from jax.experimental.pallas import tpu_sc as plsc

NC = 2
NS = 16
NW = NC * NS


def _ln(x, eps=1e-5):
    m = jnp.mean(x, axis=-1, keepdims=True)
    v = jnp.mean((x - m) ** 2, axis=-1, keepdims=True)
    return (x - m) * lax.rsqrt(v + eps)


def _dotT(x, w):
    return lax.dot_general(x, w, (((1,), (1,)), ((), ())),
                           preferred_element_type=jnp.float32)


def _pack2(a, b):
    ua = lax.bitcast_convert_type(a, jnp.uint32)
    ub = lax.bitcast_convert_type(b, jnp.uint32)
    pa = (ua + jnp.uint32(0x7FFF) + ((ua >> 16) & jnp.uint32(1))) >> 16
    pb = (ub + jnp.uint32(0x7FFF) + ((ub >> 16) & jnp.uint32(1))) >> 16
    return lax.bitcast_convert_type(pa | (pb << 16), jnp.int32)


def _unpack2(w):
    u = lax.bitcast_convert_type(w, jnp.uint32)
    a = lax.bitcast_convert_type(u << 16, jnp.float32)
    b = lax.bitcast_convert_type(u & jnp.uint32(0xFFFF0000), jnp.float32)
    return a, b


def _transform_body(x_ref, w_ref, b_ref, *o_refs):
    y = _dotT(_ln(x_ref[...]), w_ref[...]) + b_ref[...]
    h = y.shape[1] // 2
    for o_ref in o_refs:
        if o_ref.dtype == jnp.int32:
            o_ref[...] = _pack2(y[:, :h], y[:, h:])
        else:
            o_ref[...] = y.astype(o_ref.dtype)


def _transform(x, w, b, blk, dtypes=(jnp.float32,), row0=0, nrows=None):
    n, d = x.shape
    nr = n if nrows is None else nrows
    ob = row0 // blk

    def owidth(dt):
        return d // 2 if dt == jnp.int32 else d

    outs = pl.pallas_call(
        _transform_body,
        grid=(nr // blk,),
        in_specs=[pl.BlockSpec((blk, d), lambda i, ob=ob: (i + ob, 0)),
                  pl.BlockSpec((d, d), lambda i: (0, 0)),
                  pl.BlockSpec((1, d), lambda i: (0, 0))],
        out_specs=[pl.BlockSpec((blk, owidth(dt)), lambda i: (i, 0))
                   for dt in dtypes],
        out_shape=[jax.ShapeDtypeStruct((nr, owidth(dt)), dt)
                   for dt in dtypes],
    )(x, w, b.reshape(1, d))
    return outs


def _joint_body(a_ref, e_ref, c_ref, w_ref, b_ref, o_ref):
    alo, ahi = _unpack2(a_ref[...])
    elo, ehi = _unpack2(e_ref[...])
    clo, chi = _unpack2(c_ref[...])
    g = jnp.concatenate([alo + elo + clo, ahi + ehi + chi], axis=1)
    g = _ln(jnp.maximum(g, 0.0))
    o_ref[...] = _ln(_dotT(g, w_ref[...]) + b_ref[...])


def _joint(ins, w, b, blk, nrows):
    hd = ins[0][0].shape[1]
    d = 2 * hd

    def spec(off):
        ob = off // blk
        return pl.BlockSpec((blk, hd), lambda i, ob=ob: (i + ob, 0))

    return pl.pallas_call(
        _joint_body,
        grid=(nrows // blk,),
        in_specs=[spec(o) for _, o in ins] +
                 [pl.BlockSpec((d, d), lambda i: (0, 0)),
                  pl.BlockSpec((1, d), lambda i: (0, 0))],
        out_specs=pl.BlockSpec((blk, d), lambda i: (i, 0)),
        out_shape=jax.ShapeDtypeStruct((nrows, d), jnp.float32),
    )(*[a for a, _ in ins], w, b.reshape(1, d))


def _merge_body(nagg, base_ref, *rest):
    aggs = rest[:nagg]
    w_ref, b_ref = rest[nagg:nagg + 2]
    wr_ref = rest[nagg + 2] if len(rest) == nagg + 5 else None
    o_ref = rest[-2] if wr_ref is not None else rest[-1]
    d = base_ref.shape[1]
    agg = aggs[0][...]
    for a in aggs[1:]:
        agg = agg + a[...]
    h = (_dotT(base_ref[...], w_ref[:, :d]) +
         _dotT(agg, w_ref[:, d:]) + b_ref[...])
    y = base_ref[...] + _ln(jnp.maximum(h, 0.0))
    o_ref[...] = y
    if wr_ref is not None:
        z = _dotT(_ln(y), wr_ref[...])
        rest[-1][...] = _pack2(z[:, :d // 2], z[:, d // 2:])


def _merge(base, aggs, w, b, blk, w_next=None):
    n, d = base.shape
    extra_in = [] if w_next is None else [w_next]
    out_shape = [jax.ShapeDtypeStruct((n, d), jnp.float32)]
    out_specs = [pl.BlockSpec((blk, d), lambda i: (i, 0))]
    if w_next is not None:
        out_shape.append(jax.ShapeDtypeStruct((n, d // 2), jnp.int32))
        out_specs.append(pl.BlockSpec((blk, d // 2), lambda i: (i, 0)))
    res = pl.pallas_call(
        functools.partial(_merge_body, len(aggs)),
        grid=(n // blk,),
        in_specs=[pl.BlockSpec((blk, d), lambda i: (i, 0))] +
                 [pl.BlockSpec((blk, d), lambda i: (i, 0)) for _ in aggs] +
                 [pl.BlockSpec((d, 2 * d), lambda i: (0, 0)),
                  pl.BlockSpec((1, d), lambda i: (0, 0))] +
                 [pl.BlockSpec((d, d), lambda i: (0, 0))
                  for _ in extra_in],
        out_specs=out_specs,
        out_shape=out_shape,
    )(base, *aggs, w, b.reshape(1, d), *extra_in)
    return res if w_next is not None else (res[0],)


def _sc_gather(*pairs, row0=0, nrows=None):
    np_ = len(pairs)
    n, d = pairs[0][0].shape
    ne = nrows if nrows is not None else pairs[0][1].shape[0]
    dt = pairs[0][0].dtype
    chunk = 128
    nchunks = ne // chunk
    per_w = nchunks // NW
    extra = nchunks - per_w * NW
    ntask = np_ * per_w
    mesh = plsc.VectorSubcoreMesh(core_axis_name="c", subcore_axis_name="s")

    nbuf = 3

    @functools.partial(
        pl.kernel, mesh=mesh,
        out_type=[jax.ShapeDtypeStruct((ne, d), dt) for _ in pairs],
        scratch_types=[pltpu.VMEM((np_, per_w * chunk), jnp.int32),
                       pltpu.VMEM((chunk,), jnp.int32),
                       pltpu.VMEM((nbuf, chunk, d), dt)]
                      + [pltpu.SemaphoreType.DMA] * (2 * nbuf + np_),
    )
    def k(*args):
        tabs = args[:2 * np_:2]
        idxs = args[1:2 * np_:2]
        outs = args[2 * np_:3 * np_]
        idx_all, idx_x, buf_v = args[3 * np_:3 * np_ + 3]
        sems = args[3 * np_ + 3:]
        gsem, wsem, isem = sems[:nbuf], sems[nbuf:2 * nbuf], sems[2 * nbuf:]
        wid = lax.axis_index("s") * NC + lax.axis_index("c")
        ih = [pltpu.async_copy(
                  idxs[p].at[pl.ds(row0 + wid * per_w * chunk,
                                   per_w * chunk)],
                  idx_all.at[p], isem[p]) for p in range(np_)]

        def start_gather(t, sl):
            p, i = t % np_, t // np_
            return pltpu.async_copy(
                tabs[p].at[idx_all.at[p].at[pl.ds(i * chunk, chunk)]],
                buf_v.at[sl], gsem[sl])

        def start_write(t, sl):
            p, i = t % np_, t // np_
            base = (wid * per_w + i) * chunk
            return pltpu.async_copy(buf_v.at[sl],
                                    outs[p].at[pl.ds(base, chunk), :],
                                    wsem[sl])

        for h in ih:
            h.wait()
        gh, wh = {}, {}
        for j in range(min(nbuf - 1, ntask)):
            gh[j % nbuf] = start_gather(j, j % nbuf)
        for t in range(ntask):
            sl = t % nbuf
            nx = t + nbuf - 1
            if nx < ntask:
                tt = nx % nbuf
                if tt in wh:
                    wh[tt].wait()
                gh[tt] = start_gather(nx, tt)
            gh[sl].wait()
            wh[sl] = start_write(t, sl)
        for sl in wh:
            wh[sl].wait()

        if extra:
            @pl.when(wid < extra)
            def _():
                base = (NW * per_w + wid) * chunk
                for p in range(np_):
                    pltpu.sync_copy(idxs[p].at[pl.ds(row0 + base, chunk)],
                                    idx_x)
                    pltpu.async_copy(tabs[p].at[idx_x], buf_v.at[0],
                                     gsem[0]).wait()
                    pltpu.sync_copy(buf_v.at[0],
                                    outs[p].at[pl.ds(base, chunk), :])

    flat = []
    for t, i in pairs:
        flat += [t, i]
    res = k(*flat)
    return tuple(res) if isinstance(res, (list, tuple)) else (res,)


def _sc_segsum(joint, idx, nseg, row0=0):
    ne, d = joint.shape
    dh = d // NC
    chunk = 128
    nchunks = ne // chunk
    per_s = nchunks // NS
    extra = nchunks - per_s * NS
    rows_per_s = (nseg // NS) // 8 * 8
    tail = nseg - rows_per_s * NS
    zr = 16
    mesh = plsc.VectorSubcoreMesh(core_axis_name="c", subcore_axis_name="s")

    @functools.partial(
        pl.kernel, mesh=mesh,
        out_type=jax.ShapeDtypeStruct((nseg, d), jnp.float32),
        scratch_types=[pltpu.VMEM((chunk,), jnp.int32),
                       pltpu.VMEM((chunk,), jnp.int32),
                       pltpu.VMEM((2, chunk, dh), jnp.float32),
                       pltpu.VMEM((zr, dh), jnp.float32),
                       pltpu.VMEM_SHARED((nseg, dh), jnp.float32)]
                      + [pltpu.SemaphoreType.DMA] * 6,
    )
    def k(joint_hbm, idx_hbm, out_hbm, idx_a, idx_b, buf_v, zbuf,
          acc_sh, *sems):
        nbuf = 2
        c = lax.axis_index("c")
        s = lax.axis_index("s")
        idxr = (idx_a, idx_b)
        isem, rsem, ssem = sems[:2], sems[2:4], sems[4:]
        row_base = s * rows_per_s
        nz = (rows_per_s + jnp.where(s == NS - 1, tail, 0)) // zr

        def zrow(r, carry):
            def zcol(q, carry2):
                zbuf[r, pl.ds(q * 16, 16)] = jnp.zeros((16,), jnp.float32)
                return carry2
            return lax.fori_loop(0, dh // 16, zcol, carry)
        lax.fori_loop(0, zr, zrow, 0)

        def zdma(t, carry):
            pltpu.sync_copy(zbuf, acc_sh.at[pl.ds(row_base + t * zr, zr)])
            return carry
        lax.fori_loop(0, nz, zdma, 0)
        plsc.subcore_barrier()

        def start_load(i, sl):
            base = (s * per_s + i) * chunk
            ih = pltpu.async_copy(idx_hbm.at[pl.ds(row0 + base, chunk)],
                                  idxr[sl], isem[sl])
            rh = pltpu.async_copy(
                joint_hbm.at[pl.ds(base, chunk), pl.ds(c * dh, dh)],
                buf_v.at[sl], rsem[sl])
            return ih, rh

        def start_scatter(sl):
            return pltpu.async_copy(buf_v.at[sl], acc_sh.at[idxr[sl]],
                                    ssem[sl], add=True)

        lh, sh = {}, {}
        for j in range(min(nbuf - 1, per_s)):
            lh[j % nbuf] = start_load(j, j % nbuf)
        for i in range(per_s):
            sl = i % nbuf
            nx = i + nbuf - 1
            if nx < per_s:
                t = nx % nbuf
                if t in sh:
                    sh[t].wait()
                lh[t] = start_load(nx, t)
            lh[sl][0].wait()
            lh[sl][1].wait()
            sh[sl] = start_scatter(sl)
        for sl in sh:
            sh[sl].wait()

        if extra:
            @pl.when(s < extra)
            def _():
                base = (NS * per_s + s) * chunk
                pltpu.sync_copy(idx_hbm.at[pl.ds(row0 + base, chunk)], idx_a)
                pltpu.sync_copy(
                    joint_hbm.at[pl.ds(base, chunk), pl.ds(c * dh, dh)],
                    buf_v.at[0])
                pltpu.sync_copy(buf_v.at[0], acc_sh.at[idx_a], add=True)
        plsc.subcore_barrier()

        pltpu.sync_copy(
            acc_sh.at[pl.ds(row_base, rows_per_s)],
            out_hbm.at[pl.ds(row_base, rows_per_s), pl.ds(c * dh, dh)])

        if tail:
            @pl.when(s == NS - 1)
            def _():
                tb = NS * rows_per_s
                pltpu.sync_copy(
                    acc_sh.at[pl.ds(tb, tail)],
                    out_hbm.at[pl.ds(tb, tail), pl.ds(c * dh, dh)])

    return k(joint, idx)


def kernel(variable_emb, edge_emb, constraint_emb, W_left, b_left, W_edge,
           W_right, W_join, b_join, W_merge, b_merge, e_u, e_v):
    nu, d = variable_emb.shape
    nv = constraint_emb.shape[0]
    zb = jnp.zeros((d,), jnp.float32)

    pk = jnp.int32

    var_t, var_tb = _transform(variable_emb, W_left, b_left, 2000,
                               (jnp.float32, pk))
    (edge_tb,) = _transform(edge_emb, W_edge, zb, 1000, (pk,))
    con_t, con_tb = _transform(constraint_emb, W_right, zb, 2000,
                               (jnp.float32, pk))

    ne = e_u.shape[0]
    u = ne // 10
    sizes = [4 * u, 4 * u, 2 * u]
    offs = [0, 4 * u, 8 * u]
    pieces = list(zip(offs, sizes))

    et, va, ca1, agg1 = {}, {}, {}, []
    for p, (off, ln) in enumerate(pieces):
        (et[p],) = _transform(edge_emb, W_edge, zb, 1000, (pk,),
                              row0=off, nrows=ln)
        va[p], ca1[p] = _sc_gather((var_tb, e_u), (con_tb, e_v),
                                   row0=off, nrows=ln)
    for p, (off, ln) in enumerate(pieces):
        j = _joint([(va[p], 0), (et[p], 0), (ca1[p], 0)],
                   W_join, b_join, 1000, ln)
        agg1.append(_sc_segsum(j, e_v, nv, off))
    con2, con_t2b = _merge(con_t, tuple(agg1), W_merge, b_merge, 2000,
                           w_next=W_right)

    order = [2, 0, 1]
    ca2, agg2 = {}, {}
    for p in order:
        off, ln = pieces[p]
        (ca2[p],) = _sc_gather((con_t2b, e_v), row0=off, nrows=ln)
    for p in order:
        off, ln = pieces[p]
        j = _joint([(va[p], 0), (et[p], 0), (ca2[p], 0)],
                   W_join, b_join, 1000, ln)
        agg2[p] = _sc_segsum(j, e_u, nu, off)
    (var2,) = _merge(var_t, tuple(agg2[p] for p in order), W_merge,
                     b_merge, 2000)

    return (var2, con2)


# speedup vs baseline: 1.1577x; 1.1269x over previous
import functools

import jax
import jax.numpy as jnp
from jax import lax
from jax.experimental import pallas as pl
from jax.experimental.pallas import tpu as pltpu
from jax.experimental.pallas import tpu_sc as plsc

NC = 2
NS = 16
NW = NC * NS


def _ln(x, eps=1e-5):
    m = jnp.mean(x, axis=-1, keepdims=True)
    v = jnp.mean((x - m) ** 2, axis=-1, keepdims=True)
    return (x - m) * lax.rsqrt(v + eps)


def _dotT(x, w):
    return lax.dot_general(x, w, (((1,), (1,)), ((), ())),
                           preferred_element_type=jnp.float32)


def _pack2(a, b):
    ua = lax.bitcast_convert_type(a, jnp.uint32)
    ub = lax.bitcast_convert_type(b, jnp.uint32)
    pa = (ua + jnp.uint32(0x7FFF) + ((ua >> 16) & jnp.uint32(1))) >> 16
    pb = (ub + jnp.uint32(0x7FFF) + ((ub >> 16) & jnp.uint32(1))) >> 16
    return lax.bitcast_convert_type(pa | (pb << 16), jnp.int32)


def _unpack2(w):
    u = lax.bitcast_convert_type(w, jnp.uint32)
    a = lax.bitcast_convert_type(u << 16, jnp.float32)
    b = lax.bitcast_convert_type(u & jnp.uint32(0xFFFF0000), jnp.float32)
    return a, b


def _transform_body(x_ref, w_ref, b_ref, *o_refs):
    y = _dotT(_ln(x_ref[...]), w_ref[...]) + b_ref[...]
    h = y.shape[1] // 2
    for o_ref in o_refs:
        if o_ref.dtype == jnp.int32:
            o_ref[...] = _pack2(y[:, :h], y[:, h:])
        else:
            o_ref[...] = y.astype(o_ref.dtype)


def _transform(x, w, b, blk, dtypes=(jnp.float32,), row0=0, nrows=None):
    n, d = x.shape
    nr = n if nrows is None else nrows
    ob = row0 // blk

    def owidth(dt):
        return d // 2 if dt == jnp.int32 else d

    outs = pl.pallas_call(
        _transform_body,
        grid=(nr // blk,),
        in_specs=[pl.BlockSpec((blk, d), lambda i, ob=ob: (i + ob, 0)),
                  pl.BlockSpec((d, d), lambda i: (0, 0)),
                  pl.BlockSpec((1, d), lambda i: (0, 0))],
        out_specs=[pl.BlockSpec((blk, owidth(dt)), lambda i: (i, 0))
                   for dt in dtypes],
        out_shape=[jax.ShapeDtypeStruct((nr, owidth(dt)), dt)
                   for dt in dtypes],
    )(x, w, b.reshape(1, d))
    return outs


def _joint_body(a_ref, e_ref, c_ref, w_ref, b_ref, o_ref):
    alo, ahi = _unpack2(a_ref[...])
    elo, ehi = _unpack2(e_ref[...])
    clo, chi = _unpack2(c_ref[...])
    g = jnp.concatenate([alo + elo + clo, ahi + ehi + chi], axis=1)
    g = _ln(jnp.maximum(g, 0.0))
    o_ref[...] = _ln(_dotT(g, w_ref[...]) + b_ref[...])


def _joint(ins, w, b, blk, nrows):
    hd = ins[0][0].shape[1]
    d = 2 * hd

    def spec(off):
        ob = off // blk
        return pl.BlockSpec((blk, hd), lambda i, ob=ob: (i + ob, 0))

    return pl.pallas_call(
        _joint_body,
        grid=(nrows // blk,),
        in_specs=[spec(o) for _, o in ins] +
                 [pl.BlockSpec((d, d), lambda i: (0, 0)),
                  pl.BlockSpec((1, d), lambda i: (0, 0))],
        out_specs=pl.BlockSpec((blk, d), lambda i: (i, 0)),
        out_shape=jax.ShapeDtypeStruct((nrows, d), jnp.float32),
    )(*[a for a, _ in ins], w, b.reshape(1, d))


def _merge_body(nagg, base_ref, *rest):
    aggs = rest[:nagg]
    w_ref, b_ref = rest[nagg:nagg + 2]
    wr_ref = rest[nagg + 2] if len(rest) == nagg + 5 else None
    o_ref = rest[-2] if wr_ref is not None else rest[-1]
    d = base_ref.shape[1]
    agg = aggs[0][...]
    for a in aggs[1:]:
        agg = agg + a[...]
    h = (_dotT(base_ref[...], w_ref[:, :d]) +
         _dotT(agg, w_ref[:, d:]) + b_ref[...])
    y = base_ref[...] + _ln(jnp.maximum(h, 0.0))
    o_ref[...] = y
    if wr_ref is not None:
        z = _dotT(_ln(y), wr_ref[...])
        rest[-1][...] = _pack2(z[:, :d // 2], z[:, d // 2:])


def _merge(base, aggs, w, b, blk, w_next=None):
    n, d = base.shape
    extra_in = [] if w_next is None else [w_next]
    out_shape = [jax.ShapeDtypeStruct((n, d), jnp.float32)]
    out_specs = [pl.BlockSpec((blk, d), lambda i: (i, 0))]
    if w_next is not None:
        out_shape.append(jax.ShapeDtypeStruct((n, d // 2), jnp.int32))
        out_specs.append(pl.BlockSpec((blk, d // 2), lambda i: (i, 0)))
    res = pl.pallas_call(
        functools.partial(_merge_body, len(aggs)),
        grid=(n // blk,),
        in_specs=[pl.BlockSpec((blk, d), lambda i: (i, 0))] +
                 [pl.BlockSpec((blk, d), lambda i: (i, 0)) for _ in aggs] +
                 [pl.BlockSpec((d, 2 * d), lambda i: (0, 0)),
                  pl.BlockSpec((1, d), lambda i: (0, 0))] +
                 [pl.BlockSpec((d, d), lambda i: (0, 0))
                  for _ in extra_in],
        out_specs=out_specs,
        out_shape=out_shape,
    )(base, *aggs, w, b.reshape(1, d), *extra_in)
    return res if w_next is not None else (res[0],)


def _sc_gather(*pairs, row0=0, nrows=None):
    np_ = len(pairs)
    n, d = pairs[0][0].shape
    ne = nrows if nrows is not None else pairs[0][1].shape[0]
    dt = pairs[0][0].dtype
    chunk = 128
    nchunks = ne // chunk
    per_w = nchunks // NW
    extra = nchunks - per_w * NW
    ntask = np_ * per_w
    mesh = plsc.VectorSubcoreMesh(core_axis_name="c", subcore_axis_name="s")

    nbuf = 3

    @functools.partial(
        pl.kernel, mesh=mesh,
        out_type=[jax.ShapeDtypeStruct((ne, d), dt) for _ in pairs],
        scratch_types=[pltpu.VMEM((np_, per_w * chunk), jnp.int32),
                       pltpu.VMEM((chunk,), jnp.int32),
                       pltpu.VMEM((nbuf, chunk, d), dt)]
                      + [pltpu.SemaphoreType.DMA] * (2 * nbuf + np_),
    )
    def k(*args):
        tabs = args[:2 * np_:2]
        idxs = args[1:2 * np_:2]
        outs = args[2 * np_:3 * np_]
        idx_all, idx_x, buf_v = args[3 * np_:3 * np_ + 3]
        sems = args[3 * np_ + 3:]
        gsem, wsem, isem = sems[:nbuf], sems[nbuf:2 * nbuf], sems[2 * nbuf:]
        wid = lax.axis_index("s") * NC + lax.axis_index("c")
        ih = [pltpu.async_copy(
                  idxs[p].at[pl.ds(row0 + wid * per_w * chunk,
                                   per_w * chunk)],
                  idx_all.at[p], isem[p]) for p in range(np_)]

        def start_gather(t, sl):
            p, i = t % np_, t // np_
            return pltpu.async_copy(
                tabs[p].at[idx_all.at[p].at[pl.ds(i * chunk, chunk)]],
                buf_v.at[sl], gsem[sl])

        def start_write(t, sl):
            p, i = t % np_, t // np_
            base = (wid * per_w + i) * chunk
            return pltpu.async_copy(buf_v.at[sl],
                                    outs[p].at[pl.ds(base, chunk), :],
                                    wsem[sl])

        for h in ih:
            h.wait()
        gh, wh = {}, {}
        for j in range(min(nbuf - 1, ntask)):
            gh[j % nbuf] = start_gather(j, j % nbuf)
        for t in range(ntask):
            sl = t % nbuf
            nx = t + nbuf - 1
            if nx < ntask:
                tt = nx % nbuf
                if tt in wh:
                    wh[tt].wait()
                gh[tt] = start_gather(nx, tt)
            gh[sl].wait()
            wh[sl] = start_write(t, sl)
        for sl in wh:
            wh[sl].wait()

        if extra:
            @pl.when(wid < extra)
            def _():
                base = (NW * per_w + wid) * chunk
                for p in range(np_):
                    pltpu.sync_copy(idxs[p].at[pl.ds(row0 + base, chunk)],
                                    idx_x)
                    pltpu.async_copy(tabs[p].at[idx_x], buf_v.at[0],
                                     gsem[0]).wait()
                    pltpu.sync_copy(buf_v.at[0],
                                    outs[p].at[pl.ds(base, chunk), :])

    flat = []
    for t, i in pairs:
        flat += [t, i]
    res = k(*flat)
    return tuple(res) if isinstance(res, (list, tuple)) else (res,)


def _sc_segsum(joint, idx, nseg, row0=0):
    ne, d = joint.shape
    dh = d // NC
    chunk = 128
    nchunks = ne // chunk
    per_s = nchunks // NS
    extra = nchunks - per_s * NS
    rows_per_s = (nseg // NS) // 8 * 8
    tail = nseg - rows_per_s * NS
    zr = 16
    mesh = plsc.VectorSubcoreMesh(core_axis_name="c", subcore_axis_name="s")

    @functools.partial(
        pl.kernel, mesh=mesh,
        out_type=jax.ShapeDtypeStruct((nseg, d), jnp.float32),
        scratch_types=[pltpu.VMEM((chunk,), jnp.int32),
                       pltpu.VMEM((chunk,), jnp.int32),
                       pltpu.VMEM((2, chunk, dh), jnp.float32),
                       pltpu.VMEM((zr, dh), jnp.float32),
                       pltpu.VMEM_SHARED((nseg, dh), jnp.float32)]
                      + [pltpu.SemaphoreType.DMA] * 6,
    )
    def k(joint_hbm, idx_hbm, out_hbm, idx_a, idx_b, buf_v, zbuf,
          acc_sh, *sems):
        nbuf = 2
        c = lax.axis_index("c")
        s = lax.axis_index("s")
        idxr = (idx_a, idx_b)
        isem, rsem, ssem = sems[:2], sems[2:4], sems[4:]
        row_base = s * rows_per_s
        nz = (rows_per_s + jnp.where(s == NS - 1, tail, 0)) // zr

        def zrow(r, carry):
            def zcol(q, carry2):
                zbuf[r, pl.ds(q * 16, 16)] = jnp.zeros((16,), jnp.float32)
                return carry2
            return lax.fori_loop(0, dh // 16, zcol, carry)
        lax.fori_loop(0, zr, zrow, 0)

        def zdma(t, carry):
            pltpu.sync_copy(zbuf, acc_sh.at[pl.ds(row_base + t * zr, zr)])
            return carry
        lax.fori_loop(0, nz, zdma, 0)
        plsc.subcore_barrier()

        def start_load(i, sl):
            base = (s * per_s + i) * chunk
            ih = pltpu.async_copy(idx_hbm.at[pl.ds(row0 + base, chunk)],
                                  idxr[sl], isem[sl])
            rh = pltpu.async_copy(
                joint_hbm.at[pl.ds(base, chunk), pl.ds(c * dh, dh)],
                buf_v.at[sl], rsem[sl])
            return ih, rh

        def start_scatter(sl):
            return pltpu.async_copy(buf_v.at[sl], acc_sh.at[idxr[sl]],
                                    ssem[sl], add=True)

        lh, sh = {}, {}
        for j in range(min(nbuf - 1, per_s)):
            lh[j % nbuf] = start_load(j, j % nbuf)
        for i in range(per_s):
            sl = i % nbuf
            nx = i + nbuf - 1
            if nx < per_s:
                t = nx % nbuf
                if t in sh:
                    sh[t].wait()
                lh[t] = start_load(nx, t)
            lh[sl][0].wait()
            lh[sl][1].wait()
            sh[sl] = start_scatter(sl)
        for sl in sh:
            sh[sl].wait()

        if extra:
            @pl.when(s < extra)
            def _():
                base = (NS * per_s + s) * chunk
                pltpu.sync_copy(idx_hbm.at[pl.ds(row0 + base, chunk)], idx_a)
                pltpu.sync_copy(
                    joint_hbm.at[pl.ds(base, chunk), pl.ds(c * dh, dh)],
                    buf_v.at[0])
                pltpu.sync_copy(buf_v.at[0], acc_sh.at[idx_a], add=True)
        plsc.subcore_barrier()

        pltpu.sync_copy(
            acc_sh.at[pl.ds(row_base, rows_per_s)],
            out_hbm.at[pl.ds(row_base, rows_per_s), pl.ds(c * dh, dh)])

        if tail:
            @pl.when(s == NS - 1)
            def _():
                tb = NS * rows_per_s
                pltpu.sync_copy(
                    acc_sh.at[pl.ds(tb, tail)],
                    out_hbm.at[pl.ds(tb, tail), pl.ds(c * dh, dh)])

    return k(joint, idx)


def kernel(variable_emb, edge_emb, constraint_emb, W_left, b_left, W_edge,
           W_right, W_join, b_join, W_merge, b_merge, e_u, e_v):
    nu, d = variable_emb.shape
    nv = constraint_emb.shape[0]
    zb = jnp.zeros((d,), jnp.float32)

    pk = jnp.int32

    var_t, var_tb = _transform(variable_emb, W_left, b_left, 2000,
                               (jnp.float32, pk))
    con_t, con_tb = _transform(constraint_emb, W_right, zb, 2000,
                               (jnp.float32, pk))

    ne = e_u.shape[0]
    u = ne // 10
    sizes = [4 * u, 4 * u, 2 * u]
    offs = [0, 4 * u, 8 * u]
    pieces = list(zip(offs, sizes))

    et, va, ca1, agg1 = {}, {}, {}, []
    for p, (off, ln) in enumerate(pieces):
        (et[p],) = _transform(edge_emb, W_edge, zb, 2000, (pk,),
                              row0=off, nrows=ln)
        va[p], ca1[p] = _sc_gather((var_tb, e_u), (con_tb, e_v),
                                   row0=off, nrows=ln)
    for p, (off, ln) in enumerate(pieces):
        j = _joint([(va[p], 0), (et[p], 0), (ca1[p], 0)],
                   W_join, b_join, 2000, ln)
        agg1.append(_sc_segsum(j, e_v, nv, off))
    con2, con_t2b = _merge(con_t, tuple(agg1), W_merge, b_merge, 2000,
                           w_next=W_right)

    order = [2, 0, 1]
    ca2, agg2 = {}, {}
    for p in order:
        off, ln = pieces[p]
        (ca2[p],) = _sc_gather((con_t2b, e_v), row0=off, nrows=ln)
    for p in order:
        off, ln = pieces[p]
        j = _joint([(va[p], 0), (et[p], 0), (ca2[p], 0)],
                   W_join, b_join, 2000, ln)
        agg2[p] = _sc_segsum(j, e_u, nu, off)
    (var2,) = _merge(var_t, tuple(agg2[p] for p in order), W_merge,
                     b_merge, 2000)

    return (var2, con2)


# speedup vs baseline: 1.1865x; 1.0249x over previous
import functools

import jax
import jax.numpy as jnp
from jax import lax
from jax.experimental import pallas as pl
from jax.experimental.pallas import tpu as pltpu
from jax.experimental.pallas import tpu_sc as plsc

NC = 2
NS = 16
NW = NC * NS


def _ln(x, eps=1e-5):
    m = jnp.mean(x, axis=-1, keepdims=True)
    v = jnp.mean((x - m) ** 2, axis=-1, keepdims=True)
    return (x - m) * lax.rsqrt(v + eps)


def _dotT(x, w):
    return lax.dot_general(x, w, (((1,), (1,)), ((), ())),
                           preferred_element_type=jnp.float32)


def _pack2(a, b):
    ua = lax.bitcast_convert_type(a, jnp.uint32)
    ub = lax.bitcast_convert_type(b, jnp.uint32)
    pa = (ua + jnp.uint32(0x7FFF) + ((ua >> 16) & jnp.uint32(1))) >> 16
    pb = (ub + jnp.uint32(0x7FFF) + ((ub >> 16) & jnp.uint32(1))) >> 16
    return lax.bitcast_convert_type(pa | (pb << 16), jnp.int32)


def _unpack2(w):
    u = lax.bitcast_convert_type(w, jnp.uint32)
    a = lax.bitcast_convert_type(u << 16, jnp.float32)
    b = lax.bitcast_convert_type(u & jnp.uint32(0xFFFF0000), jnp.float32)
    return a, b


def _transform_body(x_ref, w_ref, b_ref, *o_refs):
    y = _dotT(_ln(x_ref[...]), w_ref[...]) + b_ref[...]
    h = y.shape[1] // 2
    for o_ref in o_refs:
        if o_ref.dtype == jnp.int32:
            o_ref[...] = _pack2(y[:, :h], y[:, h:])
        else:
            o_ref[...] = y.astype(o_ref.dtype)


def _transform(x, w, b, blk, dtypes=(jnp.float32,), row0=0, nrows=None):
    n, d = x.shape
    nr = n if nrows is None else nrows
    ob = row0 // blk

    def owidth(dt):
        return d // 2 if dt == jnp.int32 else d

    outs = pl.pallas_call(
        _transform_body,
        grid=(nr // blk,),
        in_specs=[pl.BlockSpec((blk, d), lambda i, ob=ob: (i + ob, 0)),
                  pl.BlockSpec((d, d), lambda i: (0, 0)),
                  pl.BlockSpec((1, d), lambda i: (0, 0))],
        out_specs=[pl.BlockSpec((blk, owidth(dt)), lambda i: (i, 0))
                   for dt in dtypes],
        out_shape=[jax.ShapeDtypeStruct((nr, owidth(dt)), dt)
                   for dt in dtypes],
    )(x, w, b.reshape(1, d))
    return outs


def _joint_body(a_ref, e_ref, c_ref, w_ref, b_ref, o_ref):
    alo, ahi = _unpack2(a_ref[...])
    elo, ehi = _unpack2(e_ref[...])
    clo, chi = _unpack2(c_ref[...])
    g = jnp.concatenate([alo + elo + clo, ahi + ehi + chi], axis=1)
    g = _ln(jnp.maximum(g, 0.0))
    o_ref[...] = _ln(_dotT(g, w_ref[...]) + b_ref[...])


def _joint(ins, w, b, blk, nrows):
    hd = ins[0][0].shape[1]
    d = 2 * hd

    def spec(off):
        ob = off // blk
        return pl.BlockSpec((blk, hd), lambda i, ob=ob: (i + ob, 0))

    return pl.pallas_call(
        _joint_body,
        grid=(nrows // blk,),
        in_specs=[spec(o) for _, o in ins] +
                 [pl.BlockSpec((d, d), lambda i: (0, 0)),
                  pl.BlockSpec((1, d), lambda i: (0, 0))],
        out_specs=pl.BlockSpec((blk, d), lambda i: (i, 0)),
        out_shape=jax.ShapeDtypeStruct((nrows, d), jnp.float32),
    )(*[a for a, _ in ins], w, b.reshape(1, d))


def _merge_body(nagg, base_ref, *rest):
    aggs = rest[:nagg]
    w_ref, b_ref = rest[nagg:nagg + 2]
    wr_ref = rest[nagg + 2] if len(rest) == nagg + 5 else None
    o_ref = rest[-2] if wr_ref is not None else rest[-1]
    d = base_ref.shape[1]
    agg = aggs[0][...]
    for a in aggs[1:]:
        agg = agg + a[...]
    h = (_dotT(base_ref[...], w_ref[:, :d]) +
         _dotT(agg, w_ref[:, d:]) + b_ref[...])
    y = base_ref[...] + _ln(jnp.maximum(h, 0.0))
    o_ref[...] = y
    if wr_ref is not None:
        z = _dotT(_ln(y), wr_ref[...])
        rest[-1][...] = _pack2(z[:, :d // 2], z[:, d // 2:])


def _merge(base, aggs, w, b, blk, w_next=None):
    n, d = base.shape
    extra_in = [] if w_next is None else [w_next]
    out_shape = [jax.ShapeDtypeStruct((n, d), jnp.float32)]
    out_specs = [pl.BlockSpec((blk, d), lambda i: (i, 0))]
    if w_next is not None:
        out_shape.append(jax.ShapeDtypeStruct((n, d // 2), jnp.int32))
        out_specs.append(pl.BlockSpec((blk, d // 2), lambda i: (i, 0)))
    res = pl.pallas_call(
        functools.partial(_merge_body, len(aggs)),
        grid=(n // blk,),
        in_specs=[pl.BlockSpec((blk, d), lambda i: (i, 0))] +
                 [pl.BlockSpec((blk, d), lambda i: (i, 0)) for _ in aggs] +
                 [pl.BlockSpec((d, 2 * d), lambda i: (0, 0)),
                  pl.BlockSpec((1, d), lambda i: (0, 0))] +
                 [pl.BlockSpec((d, d), lambda i: (0, 0))
                  for _ in extra_in],
        out_specs=out_specs,
        out_shape=out_shape,
    )(base, *aggs, w, b.reshape(1, d), *extra_in)
    return res if w_next is not None else (res[0],)


def _sc_gather(*pairs, row0=0, nrows=None):
    np_ = len(pairs)
    n, d = pairs[0][0].shape
    ne = nrows if nrows is not None else pairs[0][1].shape[0]
    dt = pairs[0][0].dtype
    chunk = 128
    nchunks = ne // chunk
    per_w = nchunks // NW
    extra = nchunks - per_w * NW
    ntask = np_ * per_w
    mesh = plsc.VectorSubcoreMesh(core_axis_name="c", subcore_axis_name="s")

    nbuf = 3

    @functools.partial(
        pl.kernel, mesh=mesh,
        out_type=[jax.ShapeDtypeStruct((ne, d), dt) for _ in pairs],
        scratch_types=[pltpu.VMEM((np_, per_w * chunk), jnp.int32),
                       pltpu.VMEM((chunk,), jnp.int32),
                       pltpu.VMEM((nbuf, chunk, d), dt)]
                      + [pltpu.SemaphoreType.DMA] * (2 * nbuf + np_),
    )
    def k(*args):
        tabs = args[:2 * np_:2]
        idxs = args[1:2 * np_:2]
        outs = args[2 * np_:3 * np_]
        idx_all, idx_x, buf_v = args[3 * np_:3 * np_ + 3]
        sems = args[3 * np_ + 3:]
        gsem, wsem, isem = sems[:nbuf], sems[nbuf:2 * nbuf], sems[2 * nbuf:]
        wid = lax.axis_index("s") * NC + lax.axis_index("c")
        ih = [pltpu.async_copy(
                  idxs[p].at[pl.ds(row0 + wid * per_w * chunk,
                                   per_w * chunk)],
                  idx_all.at[p], isem[p]) for p in range(np_)]

        def start_gather(t, sl):
            p, i = t % np_, t // np_
            return pltpu.async_copy(
                tabs[p].at[idx_all.at[p].at[pl.ds(i * chunk, chunk)]],
                buf_v.at[sl], gsem[sl])

        def start_write(t, sl):
            p, i = t % np_, t // np_
            base = (wid * per_w + i) * chunk
            return pltpu.async_copy(buf_v.at[sl],
                                    outs[p].at[pl.ds(base, chunk), :],
                                    wsem[sl])

        for h in ih:
            h.wait()
        gh, wh = {}, {}
        for j in range(min(nbuf - 1, ntask)):
            gh[j % nbuf] = start_gather(j, j % nbuf)
        for t in range(ntask):
            sl = t % nbuf
            nx = t + nbuf - 1
            if nx < ntask:
                tt = nx % nbuf
                if tt in wh:
                    wh[tt].wait()
                gh[tt] = start_gather(nx, tt)
            gh[sl].wait()
            wh[sl] = start_write(t, sl)
        for sl in wh:
            wh[sl].wait()

        if extra:
            @pl.when(wid < extra)
            def _():
                base = (NW * per_w + wid) * chunk
                for p in range(np_):
                    pltpu.sync_copy(idxs[p].at[pl.ds(row0 + base, chunk)],
                                    idx_x)
                    pltpu.async_copy(tabs[p].at[idx_x], buf_v.at[0],
                                     gsem[0]).wait()
                    pltpu.sync_copy(buf_v.at[0],
                                    outs[p].at[pl.ds(base, chunk), :])

    flat = []
    for t, i in pairs:
        flat += [t, i]
    res = k(*flat)
    return tuple(res) if isinstance(res, (list, tuple)) else (res,)


def _sc_segsum(joint, idx, nseg, row0=0):
    ne, d = joint.shape
    dh = d // NC
    chunk = 128
    nchunks = ne // chunk
    per_s = nchunks // NS
    extra = nchunks - per_s * NS
    rows_per_s = (nseg // NS) // 8 * 8
    tail = nseg - rows_per_s * NS
    zr = 16
    mesh = plsc.VectorSubcoreMesh(core_axis_name="c", subcore_axis_name="s")

    @functools.partial(
        pl.kernel, mesh=mesh,
        out_type=jax.ShapeDtypeStruct((nseg, d), jnp.float32),
        scratch_types=[pltpu.VMEM((chunk,), jnp.int32),
                       pltpu.VMEM((chunk,), jnp.int32),
                       pltpu.VMEM((2, chunk, dh), jnp.float32),
                       pltpu.VMEM((zr, dh), jnp.float32),
                       pltpu.VMEM_SHARED((nseg, dh), jnp.float32)]
                      + [pltpu.SemaphoreType.DMA] * 6,
    )
    def k(joint_hbm, idx_hbm, out_hbm, idx_a, idx_b, buf_v, zbuf,
          acc_sh, *sems):
        nbuf = 2
        c = lax.axis_index("c")
        s = lax.axis_index("s")
        idxr = (idx_a, idx_b)
        isem, rsem, ssem = sems[:2], sems[2:4], sems[4:]
        row_base = s * rows_per_s
        nz = (rows_per_s + jnp.where(s == NS - 1, tail, 0)) // zr

        def zrow(r, carry):
            def zcol(q, carry2):
                zbuf[r, pl.ds(q * 16, 16)] = jnp.zeros((16,), jnp.float32)
                return carry2
            return lax.fori_loop(0, dh // 16, zcol, carry)
        lax.fori_loop(0, zr, zrow, 0)

        def zdma(t, carry):
            pltpu.sync_copy(zbuf, acc_sh.at[pl.ds(row_base + t * zr, zr)])
            return carry
        lax.fori_loop(0, nz, zdma, 0)
        plsc.subcore_barrier()

        def start_load(i, sl):
            base = (s * per_s + i) * chunk
            ih = pltpu.async_copy(idx_hbm.at[pl.ds(row0 + base, chunk)],
                                  idxr[sl], isem[sl])
            rh = pltpu.async_copy(
                joint_hbm.at[pl.ds(base, chunk), pl.ds(c * dh, dh)],
                buf_v.at[sl], rsem[sl])
            return ih, rh

        def start_scatter(sl):
            return pltpu.async_copy(buf_v.at[sl], acc_sh.at[idxr[sl]],
                                    ssem[sl], add=True)

        lh, sh = {}, {}
        for j in range(min(nbuf - 1, per_s)):
            lh[j % nbuf] = start_load(j, j % nbuf)
        for i in range(per_s):
            sl = i % nbuf
            nx = i + nbuf - 1
            if nx < per_s:
                t = nx % nbuf
                if t in sh:
                    sh[t].wait()
                lh[t] = start_load(nx, t)
            lh[sl][0].wait()
            lh[sl][1].wait()
            sh[sl] = start_scatter(sl)
        for sl in sh:
            sh[sl].wait()

        if extra:
            @pl.when(s < extra)
            def _():
                base = (NS * per_s + s) * chunk
                pltpu.sync_copy(idx_hbm.at[pl.ds(row0 + base, chunk)], idx_a)
                pltpu.sync_copy(
                    joint_hbm.at[pl.ds(base, chunk), pl.ds(c * dh, dh)],
                    buf_v.at[0])
                pltpu.sync_copy(buf_v.at[0], acc_sh.at[idx_a], add=True)
        plsc.subcore_barrier()

        pltpu.sync_copy(
            acc_sh.at[pl.ds(row_base, rows_per_s)],
            out_hbm.at[pl.ds(row_base, rows_per_s), pl.ds(c * dh, dh)])

        if tail:
            @pl.when(s == NS - 1)
            def _():
                tb = NS * rows_per_s
                pltpu.sync_copy(
                    acc_sh.at[pl.ds(tb, tail)],
                    out_hbm.at[pl.ds(tb, tail), pl.ds(c * dh, dh)])

    return k(joint, idx)


def kernel(variable_emb, edge_emb, constraint_emb, W_left, b_left, W_edge,
           W_right, W_join, b_join, W_merge, b_merge, e_u, e_v):
    nu, d = variable_emb.shape
    nv = constraint_emb.shape[0]
    zb = jnp.zeros((d,), jnp.float32)

    pk = jnp.int32

    var_t, var_tb = _transform(variable_emb, W_left, b_left, 2000,
                               (jnp.float32, pk))
    con_t, con_tb = _transform(constraint_emb, W_right, zb, 2000,
                               (jnp.float32, pk))

    ne = e_u.shape[0]
    u = ne // 10
    sizes = [4 * u, 4 * u, 2 * u]
    offs = [0, 4 * u, 8 * u]
    pieces = list(zip(offs, sizes))

    et, va, ca1, agg1 = {}, {}, {}, []
    for p, (off, ln) in enumerate(pieces):
        (et[p],) = _transform(edge_emb, W_edge, zb, 4000, (pk,),
                              row0=off, nrows=ln)
        va[p], ca1[p] = _sc_gather((var_tb, e_u), (con_tb, e_v),
                                   row0=off, nrows=ln)
    for p, (off, ln) in enumerate(pieces):
        j = _joint([(va[p], 0), (et[p], 0), (ca1[p], 0)],
                   W_join, b_join, 4000, ln)
        agg1.append(_sc_segsum(j, e_v, nv, off))
    con2, con_t2b = _merge(con_t, tuple(agg1), W_merge, b_merge, 2000,
                           w_next=W_right)

    order = [2, 0, 1]
    ca2, agg2 = {}, {}
    for p in order:
        off, ln = pieces[p]
        (ca2[p],) = _sc_gather((con_t2b, e_v), row0=off, nrows=ln)
    for p in order:
        off, ln = pieces[p]
        j = _joint([(va[p], 0), (et[p], 0), (ca2[p], 0)],
                   W_join, b_join, 4000, ln)
        agg2[p] = _sc_segsum(j, e_u, nu, off)
    (var2,) = _merge(var_t, tuple(agg2[p] for p in order), W_merge,
                     b_merge, 2000)

    return (var2, con2)


# speedup vs baseline: 1.1913x; 1.0040x over previous
import functools

import jax
import jax.numpy as jnp
from jax import lax
from jax.experimental import pallas as pl
from jax.experimental.pallas import tpu as pltpu
from jax.experimental.pallas import tpu_sc as plsc

NC = 2
NS = 16
NW = NC * NS


def _ln(x, eps=1e-5):
    m = jnp.mean(x, axis=-1, keepdims=True)
    v = jnp.mean((x - m) ** 2, axis=-1, keepdims=True)
    return (x - m) * lax.rsqrt(v + eps)


def _dotT(x, w):
    return lax.dot_general(x, w, (((1,), (1,)), ((), ())),
                           preferred_element_type=jnp.float32)


def _pack2(a, b):
    ua = lax.bitcast_convert_type(a, jnp.uint32)
    ub = lax.bitcast_convert_type(b, jnp.uint32)
    pa = (ua + jnp.uint32(0x7FFF) + ((ua >> 16) & jnp.uint32(1))) >> 16
    pb = (ub + jnp.uint32(0x7FFF) + ((ub >> 16) & jnp.uint32(1))) >> 16
    return lax.bitcast_convert_type(pa | (pb << 16), jnp.int32)


def _unpack2(w):
    u = lax.bitcast_convert_type(w, jnp.uint32)
    a = lax.bitcast_convert_type(u << 16, jnp.float32)
    b = lax.bitcast_convert_type(u & jnp.uint32(0xFFFF0000), jnp.float32)
    return a, b


def _transform_body(x_ref, w_ref, b_ref, *o_refs):
    y = _dotT(_ln(x_ref[...]), w_ref[...]) + b_ref[...]
    h = y.shape[1] // 2
    for o_ref in o_refs:
        if o_ref.dtype == jnp.int32:
            o_ref[...] = _pack2(y[:, :h], y[:, h:])
        else:
            o_ref[...] = y.astype(o_ref.dtype)


def _transform(x, w, b, blk, dtypes=(jnp.float32,), row0=0, nrows=None):
    n, d = x.shape
    nr = n if nrows is None else nrows
    ob = row0 // blk

    def owidth(dt):
        return d // 2 if dt == jnp.int32 else d

    outs = pl.pallas_call(
        _transform_body,
        grid=(nr // blk,),
        in_specs=[pl.BlockSpec((blk, d), lambda i, ob=ob: (i + ob, 0)),
                  pl.BlockSpec((d, d), lambda i: (0, 0)),
                  pl.BlockSpec((1, d), lambda i: (0, 0))],
        out_specs=[pl.BlockSpec((blk, owidth(dt)), lambda i: (i, 0))
                   for dt in dtypes],
        out_shape=[jax.ShapeDtypeStruct((nr, owidth(dt)), dt)
                   for dt in dtypes],
    )(x, w, b.reshape(1, d))
    return outs


def _joint_body(a_ref, e_ref, c_ref, w_ref, b_ref, o_ref):
    alo, ahi = _unpack2(a_ref[...])
    elo, ehi = _unpack2(e_ref[...])
    clo, chi = _unpack2(c_ref[...])
    g = jnp.concatenate([alo + elo + clo, ahi + ehi + chi], axis=1)
    g = _ln(jnp.maximum(g, 0.0))
    o_ref[...] = _ln(_dotT(g, w_ref[...]) + b_ref[...])


def _joint(ins, w, b, blk, nrows):
    hd = ins[0][0].shape[1]
    d = 2 * hd

    def spec(off):
        ob = off // blk
        return pl.BlockSpec((blk, hd), lambda i, ob=ob: (i + ob, 0))

    return pl.pallas_call(
        _joint_body,
        grid=(nrows // blk,),
        in_specs=[spec(o) for _, o in ins] +
                 [pl.BlockSpec((d, d), lambda i: (0, 0)),
                  pl.BlockSpec((1, d), lambda i: (0, 0))],
        out_specs=pl.BlockSpec((blk, d), lambda i: (i, 0)),
        out_shape=jax.ShapeDtypeStruct((nrows, d), jnp.float32),
    )(*[a for a, _ in ins], w, b.reshape(1, d))


def _merge_body(nagg, base_ref, *rest):
    aggs = rest[:nagg]
    w_ref, b_ref = rest[nagg:nagg + 2]
    wr_ref = rest[nagg + 2] if len(rest) == nagg + 5 else None
    o_ref = rest[-2] if wr_ref is not None else rest[-1]
    d = base_ref.shape[1]
    agg = aggs[0][...]
    for a in aggs[1:]:
        agg = agg + a[...]
    h = (_dotT(base_ref[...], w_ref[:, :d]) +
         _dotT(agg, w_ref[:, d:]) + b_ref[...])
    y = base_ref[...] + _ln(jnp.maximum(h, 0.0))
    o_ref[...] = y
    if wr_ref is not None:
        z = _dotT(_ln(y), wr_ref[...])
        rest[-1][...] = _pack2(z[:, :d // 2], z[:, d // 2:])


def _merge(base, aggs, w, b, blk, w_next=None):
    n, d = base.shape
    extra_in = [] if w_next is None else [w_next]
    out_shape = [jax.ShapeDtypeStruct((n, d), jnp.float32)]
    out_specs = [pl.BlockSpec((blk, d), lambda i: (i, 0))]
    if w_next is not None:
        out_shape.append(jax.ShapeDtypeStruct((n, d // 2), jnp.int32))
        out_specs.append(pl.BlockSpec((blk, d // 2), lambda i: (i, 0)))
    res = pl.pallas_call(
        functools.partial(_merge_body, len(aggs)),
        grid=(n // blk,),
        in_specs=[pl.BlockSpec((blk, d), lambda i: (i, 0))] +
                 [pl.BlockSpec((blk, d), lambda i: (i, 0)) for _ in aggs] +
                 [pl.BlockSpec((d, 2 * d), lambda i: (0, 0)),
                  pl.BlockSpec((1, d), lambda i: (0, 0))] +
                 [pl.BlockSpec((d, d), lambda i: (0, 0))
                  for _ in extra_in],
        out_specs=out_specs,
        out_shape=out_shape,
    )(base, *aggs, w, b.reshape(1, d), *extra_in)
    return res if w_next is not None else (res[0],)


def _sc_gather(*pairs, row0=0, nrows=None):
    np_ = len(pairs)
    n, d = pairs[0][0].shape
    ne = nrows if nrows is not None else pairs[0][1].shape[0]
    dt = pairs[0][0].dtype
    chunk = 128
    nchunks = ne // chunk
    per_w = nchunks // NW
    extra = nchunks - per_w * NW
    ntask = np_ * per_w
    mesh = plsc.VectorSubcoreMesh(core_axis_name="c", subcore_axis_name="s")

    nbuf = 3

    @functools.partial(
        pl.kernel, mesh=mesh,
        out_type=[jax.ShapeDtypeStruct((ne, d), dt) for _ in pairs],
        scratch_types=[pltpu.VMEM((np_, per_w * chunk), jnp.int32),
                       pltpu.VMEM((chunk,), jnp.int32),
                       pltpu.VMEM((nbuf, chunk, d), dt)]
                      + [pltpu.SemaphoreType.DMA] * (2 * nbuf + np_),
    )
    def k(*args):
        tabs = args[:2 * np_:2]
        idxs = args[1:2 * np_:2]
        outs = args[2 * np_:3 * np_]
        idx_all, idx_x, buf_v = args[3 * np_:3 * np_ + 3]
        sems = args[3 * np_ + 3:]
        gsem, wsem, isem = sems[:nbuf], sems[nbuf:2 * nbuf], sems[2 * nbuf:]
        wid = lax.axis_index("s") * NC + lax.axis_index("c")
        ih = [pltpu.async_copy(
                  idxs[p].at[pl.ds(row0 + wid * per_w * chunk,
                                   per_w * chunk)],
                  idx_all.at[p], isem[p]) for p in range(np_)]

        def start_gather(t, sl):
            p, i = t % np_, t // np_
            return pltpu.async_copy(
                tabs[p].at[idx_all.at[p].at[pl.ds(i * chunk, chunk)]],
                buf_v.at[sl], gsem[sl])

        def start_write(t, sl):
            p, i = t % np_, t // np_
            base = (wid * per_w + i) * chunk
            return pltpu.async_copy(buf_v.at[sl],
                                    outs[p].at[pl.ds(base, chunk), :],
                                    wsem[sl])

        for h in ih:
            h.wait()
        gh, wh = {}, {}
        for j in range(min(nbuf - 1, ntask)):
            gh[j % nbuf] = start_gather(j, j % nbuf)
        for t in range(ntask):
            sl = t % nbuf
            nx = t + nbuf - 1
            if nx < ntask:
                tt = nx % nbuf
                if tt in wh:
                    wh[tt].wait()
                gh[tt] = start_gather(nx, tt)
            gh[sl].wait()
            wh[sl] = start_write(t, sl)
        for sl in wh:
            wh[sl].wait()

        if extra:
            @pl.when(wid < extra)
            def _():
                base = (NW * per_w + wid) * chunk
                for p in range(np_):
                    pltpu.sync_copy(idxs[p].at[pl.ds(row0 + base, chunk)],
                                    idx_x)
                    pltpu.async_copy(tabs[p].at[idx_x], buf_v.at[0],
                                     gsem[0]).wait()
                    pltpu.sync_copy(buf_v.at[0],
                                    outs[p].at[pl.ds(base, chunk), :])

    flat = []
    for t, i in pairs:
        flat += [t, i]
    res = k(*flat)
    return tuple(res) if isinstance(res, (list, tuple)) else (res,)


def _sc_segsum(joint, idx, nseg, row0=0):
    ne, d = joint.shape
    dh = d // NC
    chunk = 128
    nchunks = ne // chunk
    per_s = nchunks // NS
    extra = nchunks - per_s * NS
    rows_per_s = (nseg // NS) // 8 * 8
    tail = nseg - rows_per_s * NS
    zr = 16
    mesh = plsc.VectorSubcoreMesh(core_axis_name="c", subcore_axis_name="s")

    @functools.partial(
        pl.kernel, mesh=mesh,
        out_type=jax.ShapeDtypeStruct((nseg, d), jnp.float32),
        scratch_types=[pltpu.VMEM((chunk,), jnp.int32),
                       pltpu.VMEM((chunk,), jnp.int32),
                       pltpu.VMEM((2, chunk, dh), jnp.float32),
                       pltpu.VMEM((zr, dh), jnp.float32),
                       pltpu.VMEM_SHARED((nseg, dh), jnp.float32)]
                      + [pltpu.SemaphoreType.DMA] * 6,
    )
    def k(joint_hbm, idx_hbm, out_hbm, idx_a, idx_b, buf_v, zbuf,
          acc_sh, *sems):
        nbuf = 2
        c = lax.axis_index("c")
        s = lax.axis_index("s")
        idxr = (idx_a, idx_b)
        isem, rsem, ssem = sems[:2], sems[2:4], sems[4:]
        row_base = s * rows_per_s
        nz = (rows_per_s + jnp.where(s == NS - 1, tail, 0)) // zr

        def zrow(r, carry):
            def zcol(q, carry2):
                zbuf[r, pl.ds(q * 16, 16)] = jnp.zeros((16,), jnp.float32)
                return carry2
            return lax.fori_loop(0, dh // 16, zcol, carry)
        lax.fori_loop(0, zr, zrow, 0)

        def zdma(t, carry):
            pltpu.sync_copy(zbuf, acc_sh.at[pl.ds(row_base + t * zr, zr)])
            return carry
        lax.fori_loop(0, nz, zdma, 0)
        plsc.subcore_barrier()

        def start_load(i, sl):
            base = (s * per_s + i) * chunk
            ih = pltpu.async_copy(idx_hbm.at[pl.ds(row0 + base, chunk)],
                                  idxr[sl], isem[sl])
            rh = pltpu.async_copy(
                joint_hbm.at[pl.ds(base, chunk), pl.ds(c * dh, dh)],
                buf_v.at[sl], rsem[sl])
            return ih, rh

        def start_scatter(sl):
            return pltpu.async_copy(buf_v.at[sl], acc_sh.at[idxr[sl]],
                                    ssem[sl], add=True)

        lh, sh = {}, {}
        for j in range(min(nbuf - 1, per_s)):
            lh[j % nbuf] = start_load(j, j % nbuf)
        for i in range(per_s):
            sl = i % nbuf
            nx = i + nbuf - 1
            if nx < per_s:
                t = nx % nbuf
                if t in sh:
                    sh[t].wait()
                lh[t] = start_load(nx, t)
            lh[sl][0].wait()
            lh[sl][1].wait()
            sh[sl] = start_scatter(sl)
        for sl in sh:
            sh[sl].wait()

        if extra:
            @pl.when(s < extra)
            def _():
                base = (NS * per_s + s) * chunk
                pltpu.sync_copy(idx_hbm.at[pl.ds(row0 + base, chunk)], idx_a)
                pltpu.sync_copy(
                    joint_hbm.at[pl.ds(base, chunk), pl.ds(c * dh, dh)],
                    buf_v.at[0])
                pltpu.sync_copy(buf_v.at[0], acc_sh.at[idx_a], add=True)
        plsc.subcore_barrier()

        pltpu.sync_copy(
            acc_sh.at[pl.ds(row_base, rows_per_s)],
            out_hbm.at[pl.ds(row_base, rows_per_s), pl.ds(c * dh, dh)])

        if tail:
            @pl.when(s == NS - 1)
            def _():
                tb = NS * rows_per_s
                pltpu.sync_copy(
                    acc_sh.at[pl.ds(tb, tail)],
                    out_hbm.at[pl.ds(tb, tail), pl.ds(c * dh, dh)])

    return k(joint, idx)


def kernel(variable_emb, edge_emb, constraint_emb, W_left, b_left, W_edge,
           W_right, W_join, b_join, W_merge, b_merge, e_u, e_v):
    nu, d = variable_emb.shape
    nv = constraint_emb.shape[0]
    zb = jnp.zeros((d,), jnp.float32)

    pk = jnp.int32

    var_t, var_tb = _transform(variable_emb, W_left, b_left, 2000,
                               (jnp.float32, pk))
    con_t, con_tb = _transform(constraint_emb, W_right, zb, 2000,
                               (jnp.float32, pk))

    ne = e_u.shape[0]
    u = ne // 10
    sizes = [4 * u, 4 * u, 2 * u]
    offs = [0, 4 * u, 8 * u]
    pieces = list(zip(offs, sizes))

    et, va, ca1, agg1 = {}, {}, {}, []
    for p, (off, ln) in enumerate(pieces):
        (et[p],) = _transform(edge_emb, W_edge, zb, 8000, (pk,),
                              row0=off, nrows=ln)
        va[p], ca1[p] = _sc_gather((var_tb, e_u), (con_tb, e_v),
                                   row0=off, nrows=ln)
    for p, (off, ln) in enumerate(pieces):
        j = _joint([(va[p], 0), (et[p], 0), (ca1[p], 0)],
                   W_join, b_join, 8000, ln)
        agg1.append(_sc_segsum(j, e_v, nv, off))
    con2, con_t2b = _merge(con_t, tuple(agg1), W_merge, b_merge, 2000,
                           w_next=W_right)

    order = [2, 0, 1]
    ca2, agg2 = {}, {}
    for p in order:
        off, ln = pieces[p]
        (ca2[p],) = _sc_gather((con_t2b, e_v), row0=off, nrows=ln)
    for p in order:
        off, ln = pieces[p]
        j = _joint([(va[p], 0), (et[p], 0), (ca2[p], 0)],
                   W_join, b_join, 8000, ln)
        agg2[p] = _sc_segsum(j, e_u, nu, off)
    (var2,) = _merge(var_t, tuple(agg2[p] for p in order), W_merge,
                     b_merge, 2000)

    return (var2, con2)


# speedup vs baseline: 1.1932x; 1.0016x over previous
import functools

import jax
import jax.numpy as jnp
from jax import lax
from jax.experimental import pallas as pl
from jax.experimental.pallas import tpu as pltpu
from jax.experimental.pallas import tpu_sc as plsc

NC = 2
NS = 16
NW = NC * NS


def _ln(x, eps=1e-5):
    m = jnp.mean(x, axis=-1, keepdims=True)
    v = jnp.mean((x - m) ** 2, axis=-1, keepdims=True)
    return (x - m) * lax.rsqrt(v + eps)


def _dotT(x, w):
    return lax.dot_general(x, w, (((1,), (1,)), ((), ())),
                           preferred_element_type=jnp.float32)


def _pack2(a, b):
    ua = lax.bitcast_convert_type(a, jnp.uint32)
    ub = lax.bitcast_convert_type(b, jnp.uint32)
    pa = (ua + jnp.uint32(0x7FFF) + ((ua >> 16) & jnp.uint32(1))) >> 16
    pb = (ub + jnp.uint32(0x7FFF) + ((ub >> 16) & jnp.uint32(1))) >> 16
    return lax.bitcast_convert_type(pa | (pb << 16), jnp.int32)


def _unpack2(w):
    u = lax.bitcast_convert_type(w, jnp.uint32)
    a = lax.bitcast_convert_type(u << 16, jnp.float32)
    b = lax.bitcast_convert_type(u & jnp.uint32(0xFFFF0000), jnp.float32)
    return a, b


def _transform_body(x_ref, w_ref, b_ref, *o_refs):
    y = _dotT(_ln(x_ref[...]), w_ref[...]) + b_ref[...]
    h = y.shape[1] // 2
    for o_ref in o_refs:
        if o_ref.dtype == jnp.int32:
            o_ref[...] = _pack2(y[:, :h], y[:, h:])
        else:
            o_ref[...] = y.astype(o_ref.dtype)


def _transform(x, w, b, blk, dtypes=(jnp.float32,), row0=0, nrows=None):
    n, d = x.shape
    nr = n if nrows is None else nrows
    ob = row0 // blk

    def owidth(dt):
        return d // 2 if dt == jnp.int32 else d

    outs = pl.pallas_call(
        _transform_body,
        grid=(nr // blk,),
        in_specs=[pl.BlockSpec((blk, d), lambda i, ob=ob: (i + ob, 0)),
                  pl.BlockSpec((d, d), lambda i: (0, 0)),
                  pl.BlockSpec((1, d), lambda i: (0, 0))],
        out_specs=[pl.BlockSpec((blk, owidth(dt)), lambda i: (i, 0))
                   for dt in dtypes],
        out_shape=[jax.ShapeDtypeStruct((nr, owidth(dt)), dt)
                   for dt in dtypes],
    )(x, w, b.reshape(1, d))
    return outs


def _joint_body(a_ref, e_ref, c_ref, w_ref, b_ref, o_ref):
    alo, ahi = _unpack2(a_ref[...])
    elo, ehi = _unpack2(e_ref[...])
    clo, chi = _unpack2(c_ref[...])
    g = jnp.concatenate([alo + elo + clo, ahi + ehi + chi], axis=1)
    g = _ln(jnp.maximum(g, 0.0))
    o_ref[...] = _ln(_dotT(g, w_ref[...]) + b_ref[...])


def _joint(ins, w, b, blk, nrows):
    hd = ins[0][0].shape[1]
    d = 2 * hd

    def spec(off):
        ob = off // blk
        return pl.BlockSpec((blk, hd), lambda i, ob=ob: (i + ob, 0))

    return pl.pallas_call(
        _joint_body,
        grid=(nrows // blk,),
        in_specs=[spec(o) for _, o in ins] +
                 [pl.BlockSpec((d, d), lambda i: (0, 0)),
                  pl.BlockSpec((1, d), lambda i: (0, 0))],
        out_specs=pl.BlockSpec((blk, d), lambda i: (i, 0)),
        out_shape=jax.ShapeDtypeStruct((nrows, d), jnp.float32),
    )(*[a for a, _ in ins], w, b.reshape(1, d))


def _merge_body(nagg, base_ref, *rest):
    aggs = rest[:nagg]
    w_ref, b_ref = rest[nagg:nagg + 2]
    wr_ref = rest[nagg + 2] if len(rest) == nagg + 5 else None
    o_ref = rest[-2] if wr_ref is not None else rest[-1]
    d = base_ref.shape[1]
    agg = aggs[0][...]
    for a in aggs[1:]:
        agg = agg + a[...]
    h = (_dotT(base_ref[...], w_ref[:, :d]) +
         _dotT(agg, w_ref[:, d:]) + b_ref[...])
    y = base_ref[...] + _ln(jnp.maximum(h, 0.0))
    o_ref[...] = y
    if wr_ref is not None:
        z = _dotT(_ln(y), wr_ref[...])
        rest[-1][...] = _pack2(z[:, :d // 2], z[:, d // 2:])


def _merge(base, aggs, w, b, blk, w_next=None):
    n, d = base.shape
    extra_in = [] if w_next is None else [w_next]
    out_shape = [jax.ShapeDtypeStruct((n, d), jnp.float32)]
    out_specs = [pl.BlockSpec((blk, d), lambda i: (i, 0))]
    if w_next is not None:
        out_shape.append(jax.ShapeDtypeStruct((n, d // 2), jnp.int32))
        out_specs.append(pl.BlockSpec((blk, d // 2), lambda i: (i, 0)))
    res = pl.pallas_call(
        functools.partial(_merge_body, len(aggs)),
        grid=(n // blk,),
        in_specs=[pl.BlockSpec((blk, d), lambda i: (i, 0))] +
                 [pl.BlockSpec((blk, d), lambda i: (i, 0)) for _ in aggs] +
                 [pl.BlockSpec((d, 2 * d), lambda i: (0, 0)),
                  pl.BlockSpec((1, d), lambda i: (0, 0))] +
                 [pl.BlockSpec((d, d), lambda i: (0, 0))
                  for _ in extra_in],
        out_specs=out_specs,
        out_shape=out_shape,
    )(base, *aggs, w, b.reshape(1, d), *extra_in)
    return res if w_next is not None else (res[0],)


def _sc_gather(*pairs, row0=0, nrows=None):
    np_ = len(pairs)
    n, d = pairs[0][0].shape
    ne = nrows if nrows is not None else pairs[0][1].shape[0]
    dt = pairs[0][0].dtype
    chunk = 128
    nchunks = ne // chunk
    per_w = nchunks // NW
    extra = nchunks - per_w * NW
    ntask = np_ * per_w
    mesh = plsc.VectorSubcoreMesh(core_axis_name="c", subcore_axis_name="s")

    nbuf = 3

    @functools.partial(
        pl.kernel, mesh=mesh,
        out_type=[jax.ShapeDtypeStruct((ne, d), dt) for _ in pairs],
        scratch_types=[pltpu.VMEM((np_, per_w * chunk), jnp.int32),
                       pltpu.VMEM((chunk,), jnp.int32),
                       pltpu.VMEM((nbuf, chunk, d), dt)]
                      + [pltpu.SemaphoreType.DMA] * (2 * nbuf + np_),
    )
    def k(*args):
        tabs = args[:2 * np_:2]
        idxs = args[1:2 * np_:2]
        outs = args[2 * np_:3 * np_]
        idx_all, idx_x, buf_v = args[3 * np_:3 * np_ + 3]
        sems = args[3 * np_ + 3:]
        gsem, wsem, isem = sems[:nbuf], sems[nbuf:2 * nbuf], sems[2 * nbuf:]
        wid = lax.axis_index("s") * NC + lax.axis_index("c")
        ih = [pltpu.async_copy(
                  idxs[p].at[pl.ds(row0 + wid * per_w * chunk,
                                   per_w * chunk)],
                  idx_all.at[p], isem[p]) for p in range(np_)]

        def start_gather(t, sl):
            p, i = t % np_, t // np_
            return pltpu.async_copy(
                tabs[p].at[idx_all.at[p].at[pl.ds(i * chunk, chunk)]],
                buf_v.at[sl], gsem[sl])

        def start_write(t, sl):
            p, i = t % np_, t // np_
            base = (wid * per_w + i) * chunk
            return pltpu.async_copy(buf_v.at[sl],
                                    outs[p].at[pl.ds(base, chunk), :],
                                    wsem[sl])

        for h in ih:
            h.wait()
        gh, wh = {}, {}
        for j in range(min(nbuf - 1, ntask)):
            gh[j % nbuf] = start_gather(j, j % nbuf)
        for t in range(ntask):
            sl = t % nbuf
            nx = t + nbuf - 1
            if nx < ntask:
                tt = nx % nbuf
                if tt in wh:
                    wh[tt].wait()
                gh[tt] = start_gather(nx, tt)
            gh[sl].wait()
            wh[sl] = start_write(t, sl)
        for sl in wh:
            wh[sl].wait()

        if extra:
            @pl.when(wid < extra)
            def _():
                base = (NW * per_w + wid) * chunk
                for p in range(np_):
                    pltpu.sync_copy(idxs[p].at[pl.ds(row0 + base, chunk)],
                                    idx_x)
                    pltpu.async_copy(tabs[p].at[idx_x], buf_v.at[0],
                                     gsem[0]).wait()
                    pltpu.sync_copy(buf_v.at[0],
                                    outs[p].at[pl.ds(base, chunk), :])

    flat = []
    for t, i in pairs:
        flat += [t, i]
    res = k(*flat)
    return tuple(res) if isinstance(res, (list, tuple)) else (res,)


def _sc_segsum(joint, idx, nseg, row0=0):
    ne, d = joint.shape
    dh = d // NC
    chunk = 128
    nchunks = ne // chunk
    per_s = nchunks // NS
    extra = nchunks - per_s * NS
    rows_per_s = (nseg // NS) // 8 * 8
    tail = nseg - rows_per_s * NS
    zr = 16
    mesh = plsc.VectorSubcoreMesh(core_axis_name="c", subcore_axis_name="s")

    @functools.partial(
        pl.kernel, mesh=mesh,
        out_type=jax.ShapeDtypeStruct((nseg, d), jnp.float32),
        scratch_types=[pltpu.VMEM((chunk,), jnp.int32),
                       pltpu.VMEM((chunk,), jnp.int32),
                       pltpu.VMEM((2, chunk, dh), jnp.float32),
                       pltpu.VMEM((zr, dh), jnp.float32),
                       pltpu.VMEM_SHARED((nseg, dh), jnp.float32)]
                      + [pltpu.SemaphoreType.DMA] * 6,
    )
    def k(joint_hbm, idx_hbm, out_hbm, idx_a, idx_b, buf_v, zbuf,
          acc_sh, *sems):
        nbuf = 2
        c = lax.axis_index("c")
        s = lax.axis_index("s")
        idxr = (idx_a, idx_b)
        isem, rsem, ssem = sems[:2], sems[2:4], sems[4:]
        row_base = s * rows_per_s
        nz = (rows_per_s + jnp.where(s == NS - 1, tail, 0)) // zr

        def zrow(r, carry):
            def zcol(q, carry2):
                zbuf[r, pl.ds(q * 16, 16)] = jnp.zeros((16,), jnp.float32)
                return carry2
            return lax.fori_loop(0, dh // 16, zcol, carry)
        lax.fori_loop(0, zr, zrow, 0)

        def zdma(t, carry):
            pltpu.sync_copy(zbuf, acc_sh.at[pl.ds(row_base + t * zr, zr)])
            return carry
        lax.fori_loop(0, nz, zdma, 0)
        plsc.subcore_barrier()

        def start_load(i, sl):
            base = (s * per_s + i) * chunk
            ih = pltpu.async_copy(idx_hbm.at[pl.ds(row0 + base, chunk)],
                                  idxr[sl], isem[sl])
            rh = pltpu.async_copy(
                joint_hbm.at[pl.ds(base, chunk), pl.ds(c * dh, dh)],
                buf_v.at[sl], rsem[sl])
            return ih, rh

        def start_scatter(sl):
            return pltpu.async_copy(buf_v.at[sl], acc_sh.at[idxr[sl]],
                                    ssem[sl], add=True)

        lh, sh = {}, {}
        for j in range(min(nbuf - 1, per_s)):
            lh[j % nbuf] = start_load(j, j % nbuf)
        for i in range(per_s):
            sl = i % nbuf
            nx = i + nbuf - 1
            if nx < per_s:
                t = nx % nbuf
                if t in sh:
                    sh[t].wait()
                lh[t] = start_load(nx, t)
            lh[sl][0].wait()
            lh[sl][1].wait()
            sh[sl] = start_scatter(sl)
        for sl in sh:
            sh[sl].wait()

        if extra:
            @pl.when(s < extra)
            def _():
                base = (NS * per_s + s) * chunk
                pltpu.sync_copy(idx_hbm.at[pl.ds(row0 + base, chunk)], idx_a)
                pltpu.sync_copy(
                    joint_hbm.at[pl.ds(base, chunk), pl.ds(c * dh, dh)],
                    buf_v.at[0])
                pltpu.sync_copy(buf_v.at[0], acc_sh.at[idx_a], add=True)
        plsc.subcore_barrier()

        pltpu.sync_copy(
            acc_sh.at[pl.ds(row_base, rows_per_s)],
            out_hbm.at[pl.ds(row_base, rows_per_s), pl.ds(c * dh, dh)])

        if tail:
            @pl.when(s == NS - 1)
            def _():
                tb = NS * rows_per_s
                pltpu.sync_copy(
                    acc_sh.at[pl.ds(tb, tail)],
                    out_hbm.at[pl.ds(tb, tail), pl.ds(c * dh, dh)])

    return k(joint, idx)


def kernel(variable_emb, edge_emb, constraint_emb, W_left, b_left, W_edge,
           W_right, W_join, b_join, W_merge, b_merge, e_u, e_v):
    nu, d = variable_emb.shape
    nv = constraint_emb.shape[0]
    zb = jnp.zeros((d,), jnp.float32)

    pk = jnp.int32

    var_t, var_tb = _transform(variable_emb, W_left, b_left, 5000,
                               (jnp.float32, pk))
    con_t, con_tb = _transform(constraint_emb, W_right, zb, 5000,
                               (jnp.float32, pk))

    ne = e_u.shape[0]
    u = ne // 10
    sizes = [4 * u, 4 * u, 2 * u]
    offs = [0, 4 * u, 8 * u]
    pieces = list(zip(offs, sizes))

    et, va, ca1, agg1 = {}, {}, {}, []
    for p, (off, ln) in enumerate(pieces):
        (et[p],) = _transform(edge_emb, W_edge, zb, 8000, (pk,),
                              row0=off, nrows=ln)
        va[p], ca1[p] = _sc_gather((var_tb, e_u), (con_tb, e_v),
                                   row0=off, nrows=ln)
    for p, (off, ln) in enumerate(pieces):
        j = _joint([(va[p], 0), (et[p], 0), (ca1[p], 0)],
                   W_join, b_join, 8000, ln)
        agg1.append(_sc_segsum(j, e_v, nv, off))
    con2, con_t2b = _merge(con_t, tuple(agg1), W_merge, b_merge, 2000,
                           w_next=W_right)

    order = [2, 0, 1]
    ca2, agg2 = {}, {}
    for p in order:
        off, ln = pieces[p]
        (ca2[p],) = _sc_gather((con_t2b, e_v), row0=off, nrows=ln)
    for p in order:
        off, ln = pieces[p]
        j = _joint([(va[p], 0), (et[p], 0), (ca2[p], 0)],
                   W_join, b_join, 8000, ln)
        agg2[p] = _sc_segsum(j, e_u, nu, off)
    (var2,) = _merge(var_t, tuple(agg2[p] for p in order), W_merge,
                     b_merge, 2000)

    return (var2, con2)
